# Initial kernel scaffold; baseline (speedup 1.0000x reference)
import jax
import jax.numpy as jnp
from jax.experimental import pallas as pl


def kernel(atom_fea, nbr_fea, nbr_fea_idx, atom_spins, params):
    raise NotImplementedError("write your pallas kernel here")


# speedup vs baseline: 4.4464x; 4.4464x over previous
import functools

import jax
import jax.numpy as jnp
from jax import lax
from jax.experimental import pallas as pl
from jax.experimental.pallas import tpu as pltpu
from jax.experimental.pallas import tpu_sc as plsc

_BN = 1000
_SC_CHUNK = 128


def _softplus(x):
    return jnp.maximum(x, 0.0) + jnp.log(1.0 + jnp.exp(-jnp.abs(x)))


def _sigmoid(x):
    return 1.0 / (1.0 + jnp.exp(-x))


def _mm(a, b):
    return jnp.dot(a, b, preferred_element_type=jnp.float32)


def _layernorm(x, scale, bias, eps=1e-6):
    mu = jnp.mean(x, axis=1, keepdims=True)
    xc = x - mu
    var = jnp.mean(xc * xc, axis=1, keepdims=True)
    return xc * lax.rsqrt(var + eps) * scale + bias


def _conv_accum(af, pc, gv_ref, nbr_ref, wef, bef, wg, bg, wm, bm, M, d_e, d_a):
    acc = af
    for m in range(M):
        nb = nbr_ref[:, m * d_e:(m + 1) * d_e]
        pe = _mm(nb, wef) + bef
        inter = pc * gv_ref[:, m * d_a:(m + 1) * d_a] * pe
        gate = _sigmoid(_mm(inter, wg) + bg)
        mag = _softplus(_mm(inter, wm) + bm)
        acc = acc + gate * mag
    return acc


def _prep_body(x_ref, we_ref, be_ref, ls_ref, lb_ref, wc_ref, bc_ref,
               wn_ref, bn_ref, af0_ref, pc_ref, pn_ref):
    af0 = _mm(x_ref[...], we_ref[...]) + be_ref[...]
    an = _layernorm(af0, ls_ref[...], lb_ref[...])
    af0_ref[...] = af0
    pc_ref[...] = _mm(an, wc_ref[...]) + bc_ref[...]
    pn_ref[...] = _mm(an, wn_ref[...]) + bn_ref[...]


def _conv1_body(af_ref, pc_ref, gv_ref, nbr_ref, wef_ref, bef_ref, wg_ref,
                bg_ref, wm_ref, bm_ref, ls_ref, lb_ref, wc_ref, bc_ref,
                wn_ref, bn_ref, af1_ref, pc2_ref, pn2_ref, *, M, d_e, d_a):
    af1 = _conv_accum(af_ref[...], pc_ref[...], gv_ref, nbr_ref,
                      wef_ref[...], bef_ref[...], wg_ref[...], bg_ref[...],
                      wm_ref[...], bm_ref[...], M, d_e, d_a)
    an = _layernorm(af1, ls_ref[...], lb_ref[...])
    af1_ref[...] = af1
    pc2_ref[...] = _mm(an, wc_ref[...]) + bc_ref[...]
    pn2_ref[...] = _mm(an, wn_ref[...]) + bn_ref[...]


def _conv2_body(af_ref, pc_ref, gv_ref, nbr_ref, sp_ref, wef_ref, bef_ref,
                wg_ref, bg_ref, wm_ref, bm_ref, w1_ref, b1_ref, w2_ref,
                b2_ref, w3_ref, b3_ref, ja_ref, jb_ref, bja_ref,
                aij_ref, tbl_ref, ec_ref, *, M, d_e, d_a, d_o):
    af2 = _conv_accum(af_ref[...], pc_ref[...], gv_ref, nbr_ref,
                      wef_ref[...], bef_ref[...], wg_ref[...], bg_ref[...],
                      wm_ref[...], bm_ref[...], M, d_e, d_a)
    h = _softplus(_mm(af2, w1_ref[...]) + b1_ref[...])
    h2 = _softplus(_mm(h, w2_ref[...]) + b2_ref[...])
    ec_ref[...] = _mm(h2, w3_ref[...]) + b3_ref[...]
    aij_ref[...] = _mm(af2, ja_ref[...]) + bja_ref[...]
    tbl_ref[:, 0:d_o] = _mm(af2, jb_ref[...])
    tbl_ref[:, d_o:d_o + 1] = sp_ref[...]
    tbl_ref[:, d_o + 1:] = jnp.zeros((af2.shape[0], tbl_ref.shape[1] - d_o - 1),
                                     jnp.float32)


def _edgej_body(aij_ref, gv_ref, nbr_ref, ec_ref, sp_ref, jc_ref, j2_ref,
                bj2_ref, out_ref, *, M, d_e, d_o, wg3):
    @pl.when(pl.program_id(0) == 0)
    def _():
        out_ref[...] = jnp.zeros_like(out_ref)

    ai = aij_ref[...]
    accm = jnp.zeros((ai.shape[0], 1), jnp.float32)
    for m in range(M):
        nb = nbr_ref[:, m * d_e:(m + 1) * d_e]
        nfj = _mm(nb, jc_ref[...])
        jh = _softplus(ai + gv_ref[:, m * wg3:m * wg3 + d_o] + nfj)
        jij = _mm(jh, j2_ref[...]) + bj2_ref[...]
        sj = gv_ref[:, m * wg3 + d_o:m * wg3 + d_o + 1]
        maskf = (jnp.max(nb, axis=1, keepdims=True) > 0.0).astype(jnp.float32)
        accm = accm + jij * sj * maskf
    es = ec_ref[...] + sp_ref[...] * accm
    out_ref[...] += jnp.sum(es, axis=0, keepdims=True)


def _sc_gather(table, idx_flat):
    B, = idx_flat.shape
    V, W = table.shape
    info = plsc.get_sparse_core_info()
    nw = info.num_cores * info.num_subcores
    bpw = B // nw
    chunk = _SC_CHUNK
    nchunks = bpw // chunk
    tail = bpw - nchunks * chunk
    mesh = plsc.VectorSubcoreMesh(core_axis_name="c", subcore_axis_name="s")

    scratch = [
        pltpu.VMEM((chunk,), jnp.int32),
        pltpu.VMEM((chunk, W), jnp.float32),
        pltpu.SemaphoreType.DMA,
    ]
    if tail:
        scratch[2:2] = [
            pltpu.VMEM((tail,), jnp.int32),
            pltpu.VMEM((tail, W), jnp.float32),
        ]

    @functools.partial(
        pl.kernel, mesh=mesh,
        out_type=jax.ShapeDtypeStruct((B, W), jnp.float32),
        compiler_params=pltpu.CompilerParams(use_tc_tiling_on_sc=False),
        scratch_types=scratch,
    )
    def gather_k(table_hbm, idx_hbm, out_hbm, idx_v, rows_v, *rest):
        *tails, sem = rest
        sid = lax.axis_index("s")
        cid = lax.axis_index("c")
        wid = sid * info.num_cores + cid
        base = wid * bpw

        def body(c, carry):
            off = base + c * chunk
            pltpu.sync_copy(idx_hbm.at[pl.ds(off, chunk)], idx_v)
            pltpu.async_copy(table_hbm.at[idx_v], rows_v, sem).wait()
            pltpu.sync_copy(rows_v, out_hbm.at[pl.ds(off, chunk)])
            return carry

        lax.fori_loop(0, nchunks, body, 0)
        if tail:
            idx_t, rows_t = tails
            off = base + nchunks * chunk
            pltpu.sync_copy(idx_hbm.at[pl.ds(off, tail)], idx_t)
            pltpu.async_copy(table_hbm.at[idx_t], rows_t, sem).wait()
            pltpu.sync_copy(rows_t, out_hbm.at[pl.ds(off, tail)])

    return gather_k(table, idx_flat)


def _row_spec(d):
    return pl.BlockSpec((_BN, d), lambda i: (i, 0))


def _full_spec(arr):
    nd = arr.ndim
    return pl.BlockSpec(arr.shape, lambda i, _nd=nd: (0,) * _nd)


def _tc_call(body, grid, ins, in_row_dims, out_shapes, out_row_dims):
    in_specs = [_row_spec(d) if d is not None else _full_spec(a)
                for a, d in zip(ins, in_row_dims)]
    out_specs = [_row_spec(d) if d is not None else
                 pl.BlockSpec(s.shape, lambda i: (0, 0))
                 for s, d in zip(out_shapes, out_row_dims)]
    return pl.pallas_call(
        body,
        grid=(grid,),
        in_specs=in_specs,
        out_specs=out_specs[0] if len(out_specs) == 1 else out_specs,
        out_shape=out_shapes[0] if len(out_shapes) == 1 else out_shapes,
    )(*ins)


def kernel(atom_fea, nbr_fea, nbr_fea_idx, atom_spins, params):
    N, M = nbr_fea_idx.shape
    d_e = nbr_fea.shape[-1]
    We, be = params["embed"]
    d_a = We.shape[1]
    Wne, bne = params["nbr_embed"]
    c1, c2 = params["convs"]
    ro1w, ro1b = params["ro1"]
    ro2w, ro2b = params["ro2"]
    ro3w, ro3b = params["ro3"]
    j1w, j1b = params["J1"]
    j2w, j2b = params["J2"]
    d_o = j1w.shape[1]
    wg3 = 32

    f32 = jnp.float32
    r2 = lambda v: v.reshape(1, -1).astype(f32)

    wef1 = Wne @ c1["phi_e"][0]
    bef1 = bne @ c1["phi_e"][0] + c1["phi_e"][1]
    wef2 = Wne @ c2["phi_e"][0]
    bef2 = bne @ c2["phi_e"][0] + c2["phi_e"][1]
    j1a = j1w[:d_a]
    j1b_w = j1w[d_a:2 * d_a]
    j1c = Wne @ j1w[2 * d_a:]
    bj1f = j1b + bne @ j1w[2 * d_a:]

    nbr2d = nbr_fea.reshape(N, M * d_e)
    idx_flat = nbr_fea_idx.reshape(N * M).astype(jnp.int32)
    grid = N // _BN
    nodes = lambda d: jax.ShapeDtypeStruct((N, d), f32)

    af0, pc1, pn1 = _tc_call(
        _prep_body, grid,
        [atom_fea, We, r2(be), r2(c1["ln_scale"]), r2(c1["ln_bias"]),
         c1["phi_c"][0], r2(c1["phi_c"][1]), c1["phi_n"][0], r2(c1["phi_n"][1])],
        [atom_fea.shape[1], None, None, None, None, None, None, None, None],
        [nodes(d_a), nodes(d_a), nodes(d_a)], [d_a, d_a, d_a])

    g1 = _sc_gather(pn1, idx_flat).reshape(N, M * d_a)

    body_b = functools.partial(_conv1_body, M=M, d_e=d_e, d_a=d_a)
    af1, pc2, pn2 = _tc_call(
        body_b, grid,
        [af0, pc1, g1, nbr2d, wef1, r2(bef1),
         c1["gate"][0], r2(c1["gate"][1]), c1["mag"][0], r2(c1["mag"][1]),
         r2(c2["ln_scale"]), r2(c2["ln_bias"]),
         c2["phi_c"][0], r2(c2["phi_c"][1]), c2["phi_n"][0], r2(c2["phi_n"][1])],
        [d_a, d_a, M * d_a, M * d_e] + [None] * 12,
        [nodes(d_a), nodes(d_a), nodes(d_a)], [d_a, d_a, d_a])

    g2 = _sc_gather(pn2, idx_flat).reshape(N, M * d_a)

    body_c = functools.partial(_conv2_body, M=M, d_e=d_e, d_a=d_a, d_o=d_o)
    aij, tbl, echem = _tc_call(
        body_c, grid,
        [af1, pc2, g2, nbr2d, atom_spins, wef2, r2(bef2),
         c2["gate"][0], r2(c2["gate"][1]), c2["mag"][0], r2(c2["mag"][1]),
         ro1w, r2(ro1b), ro2w, r2(ro2b), ro3w, r2(ro3b),
         j1a, j1b_w, r2(bj1f)],
        [d_a, d_a, M * d_a, M * d_e, 1] + [None] * 15,
        [nodes(d_o), nodes(wg3), nodes(1)], [d_o, wg3, 1])

    g3 = _sc_gather(tbl, idx_flat).reshape(N, M * wg3)

    body_d = functools.partial(_edgej_body, M=M, d_e=d_e, d_o=d_o, wg3=wg3)
    total = _tc_call(
        body_d, grid,
        [aij, g3, nbr2d, echem, atom_spins, j1c, j2w, r2(j2b)],
        [d_o, M * wg3, M * d_e, 1, 1, None, None, None],
        [jax.ShapeDtypeStruct((1, 1), f32)], [None])

    return total.reshape(())


# speedup vs baseline: 5.2010x; 1.1697x over previous
import functools

import jax
import jax.numpy as jnp
from jax import lax
from jax.experimental import pallas as pl
from jax.experimental.pallas import tpu as pltpu
from jax.experimental.pallas import tpu_sc as plsc

_BN = 1000
_SC_CHUNK = 128


def _softplus(x):
    return jnp.maximum(x, 0.0) + jnp.log(1.0 + jnp.exp(-jnp.abs(x)))


def _sigmoid(x):
    return 1.0 / (1.0 + jnp.exp(-x))


def _mm(a, b):
    return jnp.dot(a, b, preferred_element_type=jnp.float32)


def _layernorm(x, scale, bias, eps=1e-6):
    mu = jnp.mean(x, axis=1, keepdims=True)
    xc = x - mu
    var = jnp.mean(xc * xc, axis=1, keepdims=True)
    return xc * lax.rsqrt(var + eps) * scale + bias


def _conv_accum(af, pc, gv_ref, nbr_ref, wef, bef, wg, bg, wm, bm, M, d_e, d_a):
    pe = _mm(nbr_ref[...], wef) + bef
    pcs = jnp.concatenate([pc] * M, axis=1)
    inter = pcs * gv_ref[...] * pe
    gate = _sigmoid(_mm(inter, wg) + bg)
    mag = _softplus(_mm(inter, wm) + bm)
    gm = gate * mag
    acc = af
    for m in range(M):
        acc = acc + gm[:, m * d_a:(m + 1) * d_a]
    return acc


def _prep_body(x_ref, we_ref, be_ref, ls_ref, lb_ref, wc_ref, bc_ref,
               wn_ref, bn_ref, af0_ref, pc_ref, pn_ref):
    af0 = _mm(x_ref[...], we_ref[...]) + be_ref[...]
    an = _layernorm(af0, ls_ref[...], lb_ref[...])
    af0_ref[...] = af0
    pc_ref[...] = _mm(an, wc_ref[...]) + bc_ref[...]
    pn_ref[...] = _mm(an, wn_ref[...]) + bn_ref[...]


def _conv1_body(af_ref, pc_ref, gv_ref, nbr_ref, wef_ref, bef_ref, wg_ref,
                bg_ref, wm_ref, bm_ref, ls_ref, lb_ref, wc_ref, bc_ref,
                wn_ref, bn_ref, af1_ref, pc2_ref, pn2_ref, *, M, d_e, d_a):
    af1 = _conv_accum(af_ref[...], pc_ref[...], gv_ref, nbr_ref,
                      wef_ref[...], bef_ref[...], wg_ref[...], bg_ref[...],
                      wm_ref[...], bm_ref[...], M, d_e, d_a)
    an = _layernorm(af1, ls_ref[...], lb_ref[...])
    af1_ref[...] = af1
    pc2_ref[...] = _mm(an, wc_ref[...]) + bc_ref[...]
    pn2_ref[...] = _mm(an, wn_ref[...]) + bn_ref[...]


def _conv2_body(af_ref, pc_ref, gv_ref, nbr_ref, sp_ref, wef_ref, bef_ref,
                wg_ref, bg_ref, wm_ref, bm_ref, w1_ref, b1_ref, w2_ref,
                b2_ref, w3_ref, b3_ref, ja_ref, jb_ref, bja_ref,
                aij_ref, tbl_ref, ec_ref, *, M, d_e, d_a, d_o):
    af2 = _conv_accum(af_ref[...], pc_ref[...], gv_ref, nbr_ref,
                      wef_ref[...], bef_ref[...], wg_ref[...], bg_ref[...],
                      wm_ref[...], bm_ref[...], M, d_e, d_a)
    h = _softplus(_mm(af2, w1_ref[...]) + b1_ref[...])
    h2 = _softplus(_mm(h, w2_ref[...]) + b2_ref[...])
    ec_ref[...] = _mm(h2, w3_ref[...]) + b3_ref[...]
    aij_ref[...] = _mm(af2, ja_ref[...]) + bja_ref[...]
    tbl_ref[:, 0:d_o] = _mm(af2, jb_ref[...])
    tbl_ref[:, d_o:d_o + 1] = sp_ref[...]
    tbl_ref[:, d_o + 1:] = jnp.zeros((af2.shape[0], tbl_ref.shape[1] - d_o - 1),
                                     jnp.float32)


def _edgej_body(aij_ref, gv_ref, nbr_ref, ec_ref, sp_ref, jc_ref, j2_ref,
                bj2_ref, out_ref, *, M, d_e, d_o, wg3):
    @pl.when(pl.program_id(0) == 0)
    def _():
        out_ref[...] = jnp.zeros_like(out_ref)

    ai = aij_ref[...]
    nfj = _mm(nbr_ref[...], jc_ref[...])
    ais = jnp.concatenate([ai] * M, axis=1)
    ajs = jnp.concatenate(
        [gv_ref[:, m * wg3:m * wg3 + d_o] for m in range(M)], axis=1)
    jh = _softplus(ais + ajs + nfj)
    jij = _mm(jh, j2_ref[...]) + bj2_ref[...]
    sj = jnp.concatenate(
        [gv_ref[:, m * wg3 + d_o:m * wg3 + d_o + 1] for m in range(M)], axis=1)
    maskf = jnp.concatenate(
        [(jnp.max(nbr_ref[:, m * d_e:(m + 1) * d_e], axis=1, keepdims=True)
          > 0.0).astype(jnp.float32) for m in range(M)], axis=1)
    accm = jnp.sum(jij * sj * maskf, axis=1, keepdims=True)
    es = ec_ref[...] + sp_ref[...] * accm
    out_ref[...] += jnp.sum(es, axis=0, keepdims=True)


def _sc_gather(table, idx_flat):
    B, = idx_flat.shape
    V, W = table.shape
    info = plsc.get_sparse_core_info()
    nw = info.num_cores * info.num_subcores
    bpw = B // nw
    chunk = _SC_CHUNK
    nchunks = bpw // chunk
    tail = bpw - nchunks * chunk
    mesh = plsc.VectorSubcoreMesh(core_axis_name="c", subcore_axis_name="s")

    scratch = [
        pltpu.VMEM((chunk,), jnp.int32),
        pltpu.VMEM((chunk, W), jnp.float32),
        pltpu.SemaphoreType.DMA,
    ]
    if tail:
        scratch[2:2] = [
            pltpu.VMEM((tail,), jnp.int32),
            pltpu.VMEM((tail, W), jnp.float32),
        ]

    @functools.partial(
        pl.kernel, mesh=mesh,
        out_type=jax.ShapeDtypeStruct((B, W), jnp.float32),
        compiler_params=pltpu.CompilerParams(use_tc_tiling_on_sc=False),
        scratch_types=scratch,
    )
    def gather_k(table_hbm, idx_hbm, out_hbm, idx_v, rows_v, *rest):
        *tails, sem = rest
        sid = lax.axis_index("s")
        cid = lax.axis_index("c")
        wid = sid * info.num_cores + cid
        base = wid * bpw

        def body(c, carry):
            off = base + c * chunk
            pltpu.sync_copy(idx_hbm.at[pl.ds(off, chunk)], idx_v)
            pltpu.async_copy(table_hbm.at[idx_v], rows_v, sem).wait()
            pltpu.sync_copy(rows_v, out_hbm.at[pl.ds(off, chunk)])
            return carry

        lax.fori_loop(0, nchunks, body, 0)
        if tail:
            idx_t, rows_t = tails
            off = base + nchunks * chunk
            pltpu.sync_copy(idx_hbm.at[pl.ds(off, tail)], idx_t)
            pltpu.async_copy(table_hbm.at[idx_t], rows_t, sem).wait()
            pltpu.sync_copy(rows_t, out_hbm.at[pl.ds(off, tail)])

    return gather_k(table, idx_flat)


def _row_spec(d):
    return pl.BlockSpec((_BN, d), lambda i: (i, 0))


def _full_spec(arr):
    nd = arr.ndim
    return pl.BlockSpec(arr.shape, lambda i, _nd=nd: (0,) * _nd)


def _tc_call(body, grid, ins, in_row_dims, out_shapes, out_row_dims):
    in_specs = [_row_spec(d) if d is not None else _full_spec(a)
                for a, d in zip(ins, in_row_dims)]
    out_specs = [_row_spec(d) if d is not None else
                 pl.BlockSpec(s.shape, lambda i: (0, 0))
                 for s, d in zip(out_shapes, out_row_dims)]
    return pl.pallas_call(
        body,
        grid=(grid,),
        in_specs=in_specs,
        out_specs=out_specs[0] if len(out_specs) == 1 else out_specs,
        out_shape=out_shapes[0] if len(out_shapes) == 1 else out_shapes,
    )(*ins)


def kernel(atom_fea, nbr_fea, nbr_fea_idx, atom_spins, params):
    N, M = nbr_fea_idx.shape
    d_e = nbr_fea.shape[-1]
    We, be = params["embed"]
    d_a = We.shape[1]
    Wne, bne = params["nbr_embed"]
    c1, c2 = params["convs"]
    ro1w, ro1b = params["ro1"]
    ro2w, ro2b = params["ro2"]
    ro3w, ro3b = params["ro3"]
    j1w, j1b = params["J1"]
    j2w, j2b = params["J2"]
    d_o = j1w.shape[1]
    wg3 = 32

    f32 = jnp.float32
    r2 = lambda v: v.reshape(1, -1).astype(f32)

    wef1 = Wne @ c1["phi_e"][0]
    bef1 = bne @ c1["phi_e"][0] + c1["phi_e"][1]
    wef2 = Wne @ c2["phi_e"][0]
    bef2 = bne @ c2["phi_e"][0] + c2["phi_e"][1]
    j1a = j1w[:d_a]
    j1b_w = j1w[d_a:2 * d_a]
    j1c = Wne @ j1w[2 * d_a:]
    bj1f = j1b + bne @ j1w[2 * d_a:]

    eyeM = jnp.eye(M, dtype=jnp.float32)
    bd = lambda w: jnp.kron(eyeM, w.astype(jnp.float32))
    tl = lambda b: jnp.tile(b.astype(jnp.float32).reshape(-1), M)
    wef1_bd, bef1_t = bd(wef1), tl(bef1)
    wg1_bd, bg1_t = bd(c1["gate"][0]), tl(c1["gate"][1])
    wm1_bd, bm1_t = bd(c1["mag"][0]), tl(c1["mag"][1])
    wef2_bd, bef2_t = bd(wef2), tl(bef2)
    wg2_bd, bg2_t = bd(c2["gate"][0]), tl(c2["gate"][1])
    wm2_bd, bm2_t = bd(c2["mag"][0]), tl(c2["mag"][1])
    jc_bd = bd(j1c)
    j2_bd = bd(j2w)
    bj2_t = tl(j2b)

    nbr2d = nbr_fea.reshape(N, M * d_e)
    idx_flat = nbr_fea_idx.reshape(N * M).astype(jnp.int32)
    grid = N // _BN
    nodes = lambda d: jax.ShapeDtypeStruct((N, d), f32)

    af0, pc1, pn1 = _tc_call(
        _prep_body, grid,
        [atom_fea, We, r2(be), r2(c1["ln_scale"]), r2(c1["ln_bias"]),
         c1["phi_c"][0], r2(c1["phi_c"][1]), c1["phi_n"][0], r2(c1["phi_n"][1])],
        [atom_fea.shape[1], None, None, None, None, None, None, None, None],
        [nodes(d_a), nodes(d_a), nodes(d_a)], [d_a, d_a, d_a])

    g1 = _sc_gather(pn1, idx_flat).reshape(N, M * d_a)

    body_b = functools.partial(_conv1_body, M=M, d_e=d_e, d_a=d_a)
    af1, pc2, pn2 = _tc_call(
        body_b, grid,
        [af0, pc1, g1, nbr2d, wef1_bd, r2(bef1_t),
         wg1_bd, r2(bg1_t), wm1_bd, r2(bm1_t),
         r2(c2["ln_scale"]), r2(c2["ln_bias"]),
         c2["phi_c"][0], r2(c2["phi_c"][1]), c2["phi_n"][0], r2(c2["phi_n"][1])],
        [d_a, d_a, M * d_a, M * d_e] + [None] * 12,
        [nodes(d_a), nodes(d_a), nodes(d_a)], [d_a, d_a, d_a])

    g2 = _sc_gather(pn2, idx_flat).reshape(N, M * d_a)

    body_c = functools.partial(_conv2_body, M=M, d_e=d_e, d_a=d_a, d_o=d_o)
    aij, tbl, echem = _tc_call(
        body_c, grid,
        [af1, pc2, g2, nbr2d, atom_spins, wef2_bd, r2(bef2_t),
         wg2_bd, r2(bg2_t), wm2_bd, r2(bm2_t),
         ro1w, r2(ro1b), ro2w, r2(ro2b), ro3w, r2(ro3b),
         j1a, j1b_w, r2(bj1f)],
        [d_a, d_a, M * d_a, M * d_e, 1] + [None] * 15,
        [nodes(d_o), nodes(wg3), nodes(1)], [d_o, wg3, 1])

    g3 = _sc_gather(tbl, idx_flat).reshape(N, M * wg3)

    body_d = functools.partial(_edgej_body, M=M, d_e=d_e, d_o=d_o, wg3=wg3)
    total = _tc_call(
        body_d, grid,
        [aij, g3, nbr2d, echem, atom_spins, jc_bd, j2_bd, r2(bj2_t)],
        [d_o, M * wg3, M * d_e, 1, 1, None, None, None],
        [jax.ShapeDtypeStruct((1, 1), f32)], [None])

    return total.reshape(())


# speedup vs baseline: 6.1621x; 1.1848x over previous
import functools

import jax
import jax.numpy as jnp
from jax import lax
from jax.experimental import pallas as pl
from jax.experimental.pallas import tpu as pltpu
from jax.experimental.pallas import tpu_sc as plsc

_BN = 1000
_SC_CHUNK = 128


def _softplus(x):
    return jnp.maximum(x, 0.0) + jnp.log(1.0 + jnp.exp(-jnp.abs(x)))


def _sigmoid(x):
    return 1.0 / (1.0 + jnp.exp(-x))


def _mm(a, b):
    return jnp.dot(a, b, preferred_element_type=jnp.float32)


def _layernorm(x, scale, bias, eps=1e-6):
    mu = jnp.mean(x, axis=1, keepdims=True)
    xc = x - mu
    var = jnp.mean(xc * xc, axis=1, keepdims=True)
    return xc * lax.rsqrt(var + eps) * scale + bias


def _conv_accum(af, pc, gv_ref, nbr_ref, wef, bef, wg, bg, wm, bm, M, d_e, d_a):
    pe = _mm(nbr_ref[...], wef) + bef
    pcs = jnp.concatenate([pc] * M, axis=1)
    inter = pcs * gv_ref[...] * pe
    gate = _sigmoid(_mm(inter, wg) + bg)
    mag = _softplus(_mm(inter, wm) + bm)
    gm = gate * mag
    acc = af
    for m in range(M):
        acc = acc + gm[:, m * d_a:(m + 1) * d_a]
    return acc


def _prep_body(x_ref, we_ref, be_ref, ls_ref, lb_ref, wc_ref, bc_ref,
               wn_ref, bn_ref, af0_ref, pc_ref, pn_ref):
    af0 = _mm(x_ref[...], we_ref[...]) + be_ref[...]
    an = _layernorm(af0, ls_ref[...], lb_ref[...])
    af0_ref[...] = af0
    pc_ref[...] = _mm(an, wc_ref[...]) + bc_ref[...]
    pn_ref[...] = _mm(an, wn_ref[...]) + bn_ref[...]


def _conv1_body(af_ref, pc_ref, gv_ref, nbr_ref, wef_ref, bef_ref, wg_ref,
                bg_ref, wm_ref, bm_ref, ls_ref, lb_ref, wc_ref, bc_ref,
                wn_ref, bn_ref, af1_ref, pc2_ref, pn2_ref, *, M, d_e, d_a):
    af1 = _conv_accum(af_ref[...], pc_ref[...], gv_ref, nbr_ref,
                      wef_ref[...], bef_ref[...], wg_ref[...], bg_ref[...],
                      wm_ref[...], bm_ref[...], M, d_e, d_a)
    an = _layernorm(af1, ls_ref[...], lb_ref[...])
    af1_ref[...] = af1
    pc2_ref[...] = _mm(an, wc_ref[...]) + bc_ref[...]
    pn2_ref[...] = _mm(an, wn_ref[...]) + bn_ref[...]


def _conv2_body(af_ref, pc_ref, gv_ref, nbr_ref, sp_ref, wef_ref, bef_ref,
                wg_ref, bg_ref, wm_ref, bm_ref, w1_ref, b1_ref, w2_ref,
                b2_ref, w3_ref, b3_ref, ja_ref, jb_ref, bja_ref,
                aij_ref, tbl_ref, ec_ref, *, M, d_e, d_a, d_o):
    af2 = _conv_accum(af_ref[...], pc_ref[...], gv_ref, nbr_ref,
                      wef_ref[...], bef_ref[...], wg_ref[...], bg_ref[...],
                      wm_ref[...], bm_ref[...], M, d_e, d_a)
    h = _softplus(_mm(af2, w1_ref[...]) + b1_ref[...])
    h2 = _softplus(_mm(h, w2_ref[...]) + b2_ref[...])
    ec_ref[...] = _mm(h2, w3_ref[...]) + b3_ref[...]
    aij_ref[...] = _mm(af2, ja_ref[...]) + bja_ref[...]
    tbl_ref[:, 0:d_o] = _mm(af2, jb_ref[...])
    tbl_ref[:, d_o:d_o + 1] = sp_ref[...]
    tbl_ref[:, d_o + 1:] = jnp.zeros((af2.shape[0], tbl_ref.shape[1] - d_o - 1),
                                     jnp.float32)


def _edgej_body(aij_ref, gv_ref, nbr_ref, ec_ref, sp_ref, jc_ref, j2_ref,
                bj2_ref, out_ref, *, M, d_e, d_o, wg3):
    @pl.when(pl.program_id(0) == 0)
    def _():
        out_ref[...] = jnp.zeros_like(out_ref)

    ai = aij_ref[...]
    nfj = _mm(nbr_ref[...], jc_ref[...])
    ais = jnp.concatenate([ai] * M, axis=1)
    ajs = jnp.concatenate(
        [gv_ref[:, m * wg3:m * wg3 + d_o] for m in range(M)], axis=1)
    jh = _softplus(ais + ajs + nfj)
    jij = _mm(jh, j2_ref[...]) + bj2_ref[...]
    sj = jnp.concatenate(
        [gv_ref[:, m * wg3 + d_o:m * wg3 + d_o + 1] for m in range(M)], axis=1)
    maskf = jnp.concatenate(
        [(jnp.max(nbr_ref[:, m * d_e:(m + 1) * d_e], axis=1, keepdims=True)
          > 0.0).astype(jnp.float32) for m in range(M)], axis=1)
    accm = jnp.sum(jij * sj * maskf, axis=1, keepdims=True)
    es = ec_ref[...] + sp_ref[...] * accm
    out_ref[...] += jnp.sum(es, axis=0, keepdims=True)


def _sc_gather(table, idx_flat):
    B, = idx_flat.shape
    V, W = table.shape
    info = plsc.get_sparse_core_info()
    nw = info.num_cores * info.num_subcores
    bpw = B // nw
    chunk = _SC_CHUNK
    nchunks = bpw // chunk
    tail = bpw - nchunks * chunk
    nhalf = nchunks // 2
    odd = nchunks - 2 * nhalf
    mesh = plsc.VectorSubcoreMesh(core_axis_name="c", subcore_axis_name="s")

    scratch = [
        pltpu.VMEM((chunk,), jnp.int32),
        pltpu.VMEM((chunk,), jnp.int32),
        pltpu.VMEM((chunk, W), jnp.float32),
        pltpu.VMEM((chunk, W), jnp.float32),
        pltpu.SemaphoreType.DMA,
        pltpu.SemaphoreType.DMA,
        pltpu.SemaphoreType.DMA,
        pltpu.SemaphoreType.DMA,
        pltpu.SemaphoreType.DMA,
        pltpu.SemaphoreType.DMA,
    ]
    if tail:
        scratch += [
            pltpu.VMEM((tail,), jnp.int32),
            pltpu.VMEM((tail, W), jnp.float32),
        ]

    @functools.partial(
        pl.kernel, mesh=mesh,
        out_type=jax.ShapeDtypeStruct((B, W), jnp.float32),
        compiler_params=pltpu.CompilerParams(use_tc_tiling_on_sc=False),
        scratch_types=scratch,
    )
    def gather_k(table_hbm, idx_hbm, out_hbm, i0, i1, r0, r1,
                 si0, si1, sg0, sg1, so0, so1, *tails):
        sid = lax.axis_index("s")
        cid = lax.axis_index("c")
        wid = sid * info.num_cores + cid
        base = wid * bpw

        def idx_start(off, ib, sem):
            pltpu.async_copy(idx_hbm.at[pl.ds(off, chunk)], ib, sem)

        def out_wait(rb, sem):
            pltpu.make_async_copy(rb, out_hbm.at[pl.ds(base, chunk)], sem).wait()

        idx_start(base, i0, si0)

        def body(c2, carry):
            off0 = base + (2 * c2) * chunk
            off1 = off0 + chunk
            pltpu.make_async_copy(idx_hbm.at[pl.ds(base, chunk)], i0, si0).wait()

            @pl.when(c2 > 0)
            def _():
                out_wait(r0, so0)

            g0 = pltpu.async_copy(table_hbm.at[i0], r0, sg0)
            idx_start(off1, i1, si1)
            g0.wait()
            pltpu.async_copy(r0, out_hbm.at[pl.ds(off0, chunk)], so0)

            pltpu.make_async_copy(idx_hbm.at[pl.ds(base, chunk)], i1, si1).wait()

            @pl.when(c2 > 0)
            def _():
                out_wait(r1, so1)

            g1 = pltpu.async_copy(table_hbm.at[i1], r1, sg1)

            @pl.when(c2 + 1 < nhalf)
            def _():
                idx_start(off1 + chunk, i0, si0)

            g1.wait()
            pltpu.async_copy(r1, out_hbm.at[pl.ds(off1, chunk)], so1)
            return carry

        lax.fori_loop(0, nhalf, body, 0)
        if odd:
            off = base + (2 * nhalf) * chunk
            pltpu.sync_copy(idx_hbm.at[pl.ds(off, chunk)], i0)
            out_wait(r0, so0)
            pltpu.async_copy(table_hbm.at[i0], r0, sg0).wait()
            pltpu.async_copy(r0, out_hbm.at[pl.ds(off, chunk)], so0)
        if tail:
            idx_t, rows_t = tails
            off = base + nchunks * chunk
            pltpu.sync_copy(idx_hbm.at[pl.ds(off, tail)], idx_t)
            pltpu.async_copy(table_hbm.at[idx_t], rows_t, sg1).wait()
            pltpu.sync_copy(rows_t, out_hbm.at[pl.ds(off, tail)])
        out_wait(r0, so0)
        out_wait(r1, so1)

    return gather_k(table, idx_flat)


def _row_spec(d):
    return pl.BlockSpec((_BN, d), lambda i: (i, 0))


def _full_spec(arr):
    nd = arr.ndim
    return pl.BlockSpec(arr.shape, lambda i, _nd=nd: (0,) * _nd)


def _tc_call(body, grid, ins, in_row_dims, out_shapes, out_row_dims):
    in_specs = [_row_spec(d) if d is not None else _full_spec(a)
                for a, d in zip(ins, in_row_dims)]
    out_specs = [_row_spec(d) if d is not None else
                 pl.BlockSpec(s.shape, lambda i: (0, 0))
                 for s, d in zip(out_shapes, out_row_dims)]
    return pl.pallas_call(
        body,
        grid=(grid,),
        in_specs=in_specs,
        out_specs=out_specs[0] if len(out_specs) == 1 else out_specs,
        out_shape=out_shapes[0] if len(out_shapes) == 1 else out_shapes,
    )(*ins)


def kernel(atom_fea, nbr_fea, nbr_fea_idx, atom_spins, params):
    N, M = nbr_fea_idx.shape
    d_e = nbr_fea.shape[-1]
    We, be = params["embed"]
    d_a = We.shape[1]
    Wne, bne = params["nbr_embed"]
    c1, c2 = params["convs"]
    ro1w, ro1b = params["ro1"]
    ro2w, ro2b = params["ro2"]
    ro3w, ro3b = params["ro3"]
    j1w, j1b = params["J1"]
    j2w, j2b = params["J2"]
    d_o = j1w.shape[1]
    wg3 = 32

    f32 = jnp.float32
    r2 = lambda v: v.reshape(1, -1).astype(f32)

    wef1 = Wne @ c1["phi_e"][0]
    bef1 = bne @ c1["phi_e"][0] + c1["phi_e"][1]
    wef2 = Wne @ c2["phi_e"][0]
    bef2 = bne @ c2["phi_e"][0] + c2["phi_e"][1]
    j1a = j1w[:d_a]
    j1b_w = j1w[d_a:2 * d_a]
    j1c = Wne @ j1w[2 * d_a:]
    bj1f = j1b + bne @ j1w[2 * d_a:]

    eyeM = jnp.eye(M, dtype=jnp.float32)
    bd = lambda w: jnp.kron(eyeM, w.astype(jnp.float32))
    tl = lambda b: jnp.tile(b.astype(jnp.float32).reshape(-1), M)
    wef1_bd, bef1_t = bd(wef1), tl(bef1)
    wg1_bd, bg1_t = bd(c1["gate"][0]), tl(c1["gate"][1])
    wm1_bd, bm1_t = bd(c1["mag"][0]), tl(c1["mag"][1])
    wef2_bd, bef2_t = bd(wef2), tl(bef2)
    wg2_bd, bg2_t = bd(c2["gate"][0]), tl(c2["gate"][1])
    wm2_bd, bm2_t = bd(c2["mag"][0]), tl(c2["mag"][1])
    jc_bd = bd(j1c)
    j2_bd = bd(j2w)
    bj2_t = tl(j2b)

    nbr2d = nbr_fea.reshape(N, M * d_e)
    idx_flat = nbr_fea_idx.reshape(N * M).astype(jnp.int32)
    grid = N // _BN
    nodes = lambda d: jax.ShapeDtypeStruct((N, d), f32)

    af0, pc1, pn1 = _tc_call(
        _prep_body, grid,
        [atom_fea, We, r2(be), r2(c1["ln_scale"]), r2(c1["ln_bias"]),
         c1["phi_c"][0], r2(c1["phi_c"][1]), c1["phi_n"][0], r2(c1["phi_n"][1])],
        [atom_fea.shape[1], None, None, None, None, None, None, None, None],
        [nodes(d_a), nodes(d_a), nodes(d_a)], [d_a, d_a, d_a])

    g1 = _sc_gather(pn1, idx_flat).reshape(N, M * d_a)

    body_b = functools.partial(_conv1_body, M=M, d_e=d_e, d_a=d_a)
    af1, pc2, pn2 = _tc_call(
        body_b, grid,
        [af0, pc1, g1, nbr2d, wef1_bd, r2(bef1_t),
         wg1_bd, r2(bg1_t), wm1_bd, r2(bm1_t),
         r2(c2["ln_scale"]), r2(c2["ln_bias"]),
         c2["phi_c"][0], r2(c2["phi_c"][1]), c2["phi_n"][0], r2(c2["phi_n"][1])],
        [d_a, d_a, M * d_a, M * d_e] + [None] * 12,
        [nodes(d_a), nodes(d_a), nodes(d_a)], [d_a, d_a, d_a])

    g2 = _sc_gather(pn2, idx_flat).reshape(N, M * d_a)

    body_c = functools.partial(_conv2_body, M=M, d_e=d_e, d_a=d_a, d_o=d_o)
    aij, tbl, echem = _tc_call(
        body_c, grid,
        [af1, pc2, g2, nbr2d, atom_spins, wef2_bd, r2(bef2_t),
         wg2_bd, r2(bg2_t), wm2_bd, r2(bm2_t),
         ro1w, r2(ro1b), ro2w, r2(ro2b), ro3w, r2(ro3b),
         j1a, j1b_w, r2(bj1f)],
        [d_a, d_a, M * d_a, M * d_e, 1] + [None] * 15,
        [nodes(d_o), nodes(wg3), nodes(1)], [d_o, wg3, 1])

    g3 = _sc_gather(tbl, idx_flat).reshape(N, M * wg3)

    body_d = functools.partial(_edgej_body, M=M, d_e=d_e, d_o=d_o, wg3=wg3)
    total = _tc_call(
        body_d, grid,
        [aij, g3, nbr2d, echem, atom_spins, jc_bd, j2_bd, r2(bj2_t)],
        [d_o, M * wg3, M * d_e, 1, 1, None, None, None],
        [jax.ShapeDtypeStruct((1, 1), f32)], [None])

    return total.reshape(())


# speedup vs baseline: 7.8457x; 1.2732x over previous
import functools

import jax
import jax.numpy as jnp
from jax import lax
from jax.experimental import pallas as pl
from jax.experimental.pallas import tpu as pltpu
from jax.experimental.pallas import tpu_sc as plsc

_BN = 1000
_SC_CHUNK = 128


def _softplus(x):
    return jnp.maximum(x, 0.0) + jnp.log(1.0 + jnp.exp(-jnp.abs(x)))


def _sigmoid(x):
    return 1.0 / (1.0 + jnp.exp(-x))


def _mm(a, b):
    return jnp.dot(a, b, preferred_element_type=jnp.float32)


def _layernorm(x, scale, bias, eps=1e-6):
    mu = jnp.mean(x, axis=1, keepdims=True)
    xc = x - mu
    var = jnp.mean(xc * xc, axis=1, keepdims=True)
    return xc * lax.rsqrt(var + eps) * scale + bias


def _conv_accum(af, pc, gv_ref, nbr_ref, wef, bef, wg, bg, wm, bm, tileT,
                sumT):
    pe = _mm(nbr_ref[...], wef) + bef
    pcs = _mm(pc, tileT)
    inter = pcs * gv_ref[...] * pe
    gate = _sigmoid(_mm(inter, wg) + bg)
    mag = _softplus(_mm(inter, wm) + bm)
    return af + _mm(gate * mag, sumT)


def _prep_body(x_ref, we_ref, be_ref, ls_ref, lb_ref, wc_ref, bc_ref,
               wn_ref, bn_ref, af0_ref, pc_ref, pn_ref):
    af0 = _mm(x_ref[...], we_ref[...]) + be_ref[...]
    an = _layernorm(af0, ls_ref[...], lb_ref[...])
    af0_ref[...] = af0
    pc_ref[...] = _mm(an, wc_ref[...]) + bc_ref[...]
    pn_ref[...] = _mm(an, wn_ref[...]) + bn_ref[...]


def _conv1_body(af_ref, pc_ref, gv_ref, nbr_ref, wef_ref, bef_ref, wg_ref,
                bg_ref, wm_ref, bm_ref, tt_ref, st_ref, ls_ref, lb_ref,
                wc_ref, bc_ref, wn_ref, bn_ref, af1_ref, pc2_ref, pn2_ref):
    af1 = _conv_accum(af_ref[...], pc_ref[...], gv_ref, nbr_ref,
                      wef_ref[...], bef_ref[...], wg_ref[...], bg_ref[...],
                      wm_ref[...], bm_ref[...], tt_ref[...], st_ref[...])
    an = _layernorm(af1, ls_ref[...], lb_ref[...])
    af1_ref[...] = af1
    pc2_ref[...] = _mm(an, wc_ref[...]) + bc_ref[...]
    pn2_ref[...] = _mm(an, wn_ref[...]) + bn_ref[...]


def _conv2_body(af_ref, pc_ref, gv_ref, nbr_ref, sp_ref, wef_ref, bef_ref,
                wg_ref, bg_ref, wm_ref, bm_ref, tt_ref, st_ref, w1_ref,
                b1_ref, w2_ref, b2_ref, w3_ref, b3_ref, ja_ref, jb_ref,
                bja_ref, aij_ref, tbl_ref, ec_ref, *, d_o):
    af2 = _conv_accum(af_ref[...], pc_ref[...], gv_ref, nbr_ref,
                      wef_ref[...], bef_ref[...], wg_ref[...], bg_ref[...],
                      wm_ref[...], bm_ref[...], tt_ref[...], st_ref[...])
    h = _softplus(_mm(af2, w1_ref[...]) + b1_ref[...])
    h2 = _softplus(_mm(h, w2_ref[...]) + b2_ref[...])
    ec_ref[...] = _mm(h2, w3_ref[...]) + b3_ref[...]
    aij_ref[...] = _mm(af2, ja_ref[...]) + bja_ref[...]
    tbl_ref[:, 0:d_o] = _mm(af2, jb_ref[...])
    tbl_ref[:, d_o:] = sp_ref[...] * jnp.ones((1, d_o), jnp.float32)


def _edgej_body(aij_ref, gv_ref, nbr_ref, ec_ref, sp_ref, tai_ref, jcf_ref,
                wp_ref, ws_ref, wones_ref, bj2_ref, out_ref):
    @pl.when(pl.program_id(0) == 0)
    def _():
        out_ref[...] = jnp.zeros_like(out_ref)

    mh = (_mm(aij_ref[...], tai_ref[...]) + gv_ref[...]
          + _mm(nbr_ref[...], jcf_ref[...]))
    jh = _softplus(mh)
    jij = _mm(jh, wp_ref[...]) + bj2_ref[...]
    sj = _mm(mh, ws_ref[...])
    maskf = (_mm(nbr_ref[...], wones_ref[...]) > 0.0).astype(jnp.float32)
    accm = jnp.sum(jij * sj * maskf, axis=1, keepdims=True)
    es = ec_ref[...] + sp_ref[...] * accm
    out_ref[...] += jnp.sum(es, axis=0, keepdims=True)


def _sc_gather(table, idx_flat):
    B, = idx_flat.shape
    V, W = table.shape
    info = plsc.get_sparse_core_info()
    nw = info.num_cores * info.num_subcores
    bpw = B // nw
    chunk = _SC_CHUNK
    nchunks = bpw // chunk
    tail = bpw - nchunks * chunk
    nhalf = nchunks // 2
    odd = nchunks - 2 * nhalf
    mesh = plsc.VectorSubcoreMesh(core_axis_name="c", subcore_axis_name="s")

    scratch = [
        pltpu.VMEM((chunk,), jnp.int32),
        pltpu.VMEM((chunk,), jnp.int32),
        pltpu.VMEM((chunk, W), jnp.float32),
        pltpu.VMEM((chunk, W), jnp.float32),
        pltpu.SemaphoreType.DMA,
        pltpu.SemaphoreType.DMA,
        pltpu.SemaphoreType.DMA,
        pltpu.SemaphoreType.DMA,
        pltpu.SemaphoreType.DMA,
        pltpu.SemaphoreType.DMA,
    ]
    if tail:
        scratch += [
            pltpu.VMEM((tail,), jnp.int32),
            pltpu.VMEM((tail, W), jnp.float32),
        ]

    @functools.partial(
        pl.kernel, mesh=mesh,
        out_type=jax.ShapeDtypeStruct((B, W), jnp.float32),
        compiler_params=pltpu.CompilerParams(use_tc_tiling_on_sc=False),
        scratch_types=scratch,
    )
    def gather_k(table_hbm, idx_hbm, out_hbm, i0, i1, r0, r1,
                 si0, si1, sg0, sg1, so0, so1, *tails):
        sid = lax.axis_index("s")
        cid = lax.axis_index("c")
        wid = sid * info.num_cores + cid
        base = wid * bpw

        def idx_start(off, ib, sem):
            pltpu.async_copy(idx_hbm.at[pl.ds(off, chunk)], ib, sem)

        def out_wait(rb, sem):
            pltpu.make_async_copy(rb, out_hbm.at[pl.ds(base, chunk)], sem).wait()

        idx_start(base, i0, si0)

        def body(c2, carry):
            off0 = base + (2 * c2) * chunk
            off1 = off0 + chunk
            pltpu.make_async_copy(idx_hbm.at[pl.ds(base, chunk)], i0, si0).wait()

            @pl.when(c2 > 0)
            def _():
                out_wait(r0, so0)

            g0 = pltpu.async_copy(table_hbm.at[i0], r0, sg0)
            idx_start(off1, i1, si1)
            g0.wait()
            pltpu.async_copy(r0, out_hbm.at[pl.ds(off0, chunk)], so0)

            pltpu.make_async_copy(idx_hbm.at[pl.ds(base, chunk)], i1, si1).wait()

            @pl.when(c2 > 0)
            def _():
                out_wait(r1, so1)

            g1 = pltpu.async_copy(table_hbm.at[i1], r1, sg1)

            @pl.when(c2 + 1 < nhalf)
            def _():
                idx_start(off1 + chunk, i0, si0)

            g1.wait()
            pltpu.async_copy(r1, out_hbm.at[pl.ds(off1, chunk)], so1)
            return carry

        lax.fori_loop(0, nhalf, body, 0)
        if odd:
            off = base + (2 * nhalf) * chunk
            pltpu.sync_copy(idx_hbm.at[pl.ds(off, chunk)], i0)
            out_wait(r0, so0)
            pltpu.async_copy(table_hbm.at[i0], r0, sg0).wait()
            pltpu.async_copy(r0, out_hbm.at[pl.ds(off, chunk)], so0)
        if tail:
            idx_t, rows_t = tails
            off = base + nchunks * chunk
            pltpu.sync_copy(idx_hbm.at[pl.ds(off, tail)], idx_t)
            pltpu.async_copy(table_hbm.at[idx_t], rows_t, sg1).wait()
            pltpu.sync_copy(rows_t, out_hbm.at[pl.ds(off, tail)])
        out_wait(r0, so0)
        out_wait(r1, so1)

    return gather_k(table, idx_flat)


def _row_spec(d):
    return pl.BlockSpec((_BN, d), lambda i: (i, 0))


def _full_spec(arr):
    nd = arr.ndim
    return pl.BlockSpec(arr.shape, lambda i, _nd=nd: (0,) * _nd)


def _tc_call(body, grid, ins, in_row_dims, out_shapes, out_row_dims):
    in_specs = [_row_spec(d) if d is not None else _full_spec(a)
                for a, d in zip(ins, in_row_dims)]
    out_specs = [_row_spec(d) if d is not None else
                 pl.BlockSpec(s.shape, lambda i: (0, 0))
                 for s, d in zip(out_shapes, out_row_dims)]
    return pl.pallas_call(
        body,
        grid=(grid,),
        in_specs=in_specs,
        out_specs=out_specs[0] if len(out_specs) == 1 else out_specs,
        out_shape=out_shapes[0] if len(out_shapes) == 1 else out_shapes,
    )(*ins)


def kernel(atom_fea, nbr_fea, nbr_fea_idx, atom_spins, params):
    N, M = nbr_fea_idx.shape
    d_e = nbr_fea.shape[-1]
    We, be = params["embed"]
    d_a = We.shape[1]
    Wne, bne = params["nbr_embed"]
    c1, c2 = params["convs"]
    ro1w, ro1b = params["ro1"]
    ro2w, ro2b = params["ro2"]
    ro3w, ro3b = params["ro3"]
    j1w, j1b = params["J1"]
    j2w, j2b = params["J2"]
    d_o = j1w.shape[1]
    wg3 = 2 * d_o

    f32 = jnp.float32
    r2 = lambda v: v.reshape(1, -1).astype(f32)

    wef1 = Wne @ c1["phi_e"][0]
    bef1 = bne @ c1["phi_e"][0] + c1["phi_e"][1]
    wef2 = Wne @ c2["phi_e"][0]
    bef2 = bne @ c2["phi_e"][0] + c2["phi_e"][1]
    j1a = j1w[:d_a]
    j1b_w = j1w[d_a:2 * d_a]
    j1c = Wne @ j1w[2 * d_a:]
    bj1f = j1b + bne @ j1w[2 * d_a:]

    eyeM = jnp.eye(M, dtype=jnp.float32)
    onesM = jnp.ones((1, M), jnp.float32)
    bd = lambda w: jnp.kron(eyeM, w.astype(jnp.float32))
    tl = lambda b: jnp.tile(b.astype(jnp.float32).reshape(-1), M)
    wef1_bd, bef1_t = bd(wef1), tl(bef1)
    wg1_bd, bg1_t = bd(c1["gate"][0]), tl(c1["gate"][1])
    wm1_bd, bm1_t = bd(c1["mag"][0]), tl(c1["mag"][1])
    wef2_bd, bef2_t = bd(wef2), tl(bef2)
    wg2_bd, bg2_t = bd(c2["gate"][0]), tl(c2["gate"][1])
    wm2_bd, bm2_t = bd(c2["mag"][0]), tl(c2["mag"][1])
    eyeA = jnp.eye(d_a, dtype=jnp.float32)
    tileT = jnp.kron(onesM, eyeA)
    sumT = jnp.kron(onesM.T, eyeA)
    zdo = jnp.zeros((d_o, d_o), jnp.float32)
    taiT = jnp.kron(onesM, jnp.concatenate([jnp.eye(d_o), zdo], 1))
    jcf_bd = bd(jnp.concatenate([j1c, jnp.zeros((d_e, d_o))], 1))
    wp_bd = bd(jnp.concatenate([j2w, jnp.zeros((d_o, 1))], 0))
    ws_bd = bd(jnp.concatenate([jnp.zeros((d_o, 1)),
                                jnp.full((d_o, 1), 1.0 / d_o)], 0))
    wones_bd = bd(jnp.ones((d_e, 1), jnp.float32))
    bj2_t = tl(j2b)

    nbr2d = nbr_fea.reshape(N, M * d_e)
    idx_flat = nbr_fea_idx.reshape(N * M).astype(jnp.int32)
    grid = N // _BN
    nodes = lambda d: jax.ShapeDtypeStruct((N, d), f32)

    af0, pc1, pn1 = _tc_call(
        _prep_body, grid,
        [atom_fea, We, r2(be), r2(c1["ln_scale"]), r2(c1["ln_bias"]),
         c1["phi_c"][0], r2(c1["phi_c"][1]), c1["phi_n"][0], r2(c1["phi_n"][1])],
        [atom_fea.shape[1], None, None, None, None, None, None, None, None],
        [nodes(d_a), nodes(d_a), nodes(d_a)], [d_a, d_a, d_a])

    g1 = _sc_gather(pn1, idx_flat).reshape(N, M * d_a)

    af1, pc2, pn2 = _tc_call(
        _conv1_body, grid,
        [af0, pc1, g1, nbr2d, wef1_bd, r2(bef1_t),
         wg1_bd, r2(bg1_t), wm1_bd, r2(bm1_t), tileT, sumT,
         r2(c2["ln_scale"]), r2(c2["ln_bias"]),
         c2["phi_c"][0], r2(c2["phi_c"][1]), c2["phi_n"][0], r2(c2["phi_n"][1])],
        [d_a, d_a, M * d_a, M * d_e] + [None] * 14,
        [nodes(d_a), nodes(d_a), nodes(d_a)], [d_a, d_a, d_a])

    g2 = _sc_gather(pn2, idx_flat).reshape(N, M * d_a)

    body_c = functools.partial(_conv2_body, d_o=d_o)
    aij, tbl, echem = _tc_call(
        body_c, grid,
        [af1, pc2, g2, nbr2d, atom_spins, wef2_bd, r2(bef2_t),
         wg2_bd, r2(bg2_t), wm2_bd, r2(bm2_t), tileT, sumT,
         ro1w, r2(ro1b), ro2w, r2(ro2b), ro3w, r2(ro3b),
         j1a, j1b_w, r2(bj1f)],
        [d_a, d_a, M * d_a, M * d_e, 1] + [None] * 17,
        [nodes(d_o), nodes(wg3), nodes(1)], [d_o, wg3, 1])

    g3 = _sc_gather(tbl, idx_flat).reshape(N, M * wg3)

    total = _tc_call(
        _edgej_body, grid,
        [aij, g3, nbr2d, echem, atom_spins, taiT, jcf_bd, wp_bd, ws_bd,
         wones_bd, r2(bj2_t)],
        [d_o, M * wg3, M * d_e, 1, 1] + [None] * 6,
        [jax.ShapeDtypeStruct((1, 1), f32)], [None])

    return total.reshape(())


# speedup vs baseline: 8.8642x; 1.1298x over previous
import functools

import jax
import jax.numpy as jnp
from jax import lax
from jax.experimental import pallas as pl
from jax.experimental.pallas import tpu as pltpu
from jax.experimental.pallas import tpu_sc as plsc

_BN = 1000
_SC_CHUNK = 128


def _softplus(x):
    return jnp.maximum(x, 0.0) + jnp.log(1.0 + jnp.exp(-jnp.abs(x)))


def _sigmoid(x):
    return 1.0 / (1.0 + jnp.exp(-x))


def _mm(a, b):
    return jnp.dot(a, b, preferred_element_type=jnp.float32)


def _layernorm(x, scale, bias, eps=1e-6):
    mu = jnp.mean(x, axis=1, keepdims=True)
    xc = x - mu
    var = jnp.mean(xc * xc, axis=1, keepdims=True)
    return xc * lax.rsqrt(var + eps) * scale + bias


def _conv_accum(af, pc, gv_ref, nbr_ref, wef, bef, wg, bg, wm, bm, tileT,
                sumT):
    pe = _mm(nbr_ref[...], wef) + bef
    pcs = _mm(pc, tileT)
    inter = pcs * gv_ref[...] * pe
    gate = _sigmoid(_mm(inter, wg) + bg)
    mag = _softplus(_mm(inter, wm) + bm)
    return af + _mm(gate * mag, sumT)


def _prep_body(x_ref, we_ref, be_ref, ls_ref, lb_ref, wc_ref, bc_ref,
               wn_ref, bn_ref, af0_ref, pc_ref, pn_ref):
    af0 = _mm(x_ref[...], we_ref[...]) + be_ref[...]
    an = _layernorm(af0, ls_ref[...], lb_ref[...])
    af0_ref[...] = af0
    pc_ref[...] = _mm(an, wc_ref[...]) + bc_ref[...]
    pn_ref[...] = _mm(an, wn_ref[...]) + bn_ref[...]


def _conv1_body(af_ref, pc_ref, gv_ref, nbr_ref, wef_ref, bef_ref, wg_ref,
                bg_ref, wm_ref, bm_ref, tt_ref, st_ref, ls_ref, lb_ref,
                wc_ref, bc_ref, wn_ref, bn_ref, af1_ref, pc2_ref, pn2_ref):
    af1 = _conv_accum(af_ref[...], pc_ref[...], gv_ref, nbr_ref,
                      wef_ref[...], bef_ref[...], wg_ref[...], bg_ref[...],
                      wm_ref[...], bm_ref[...], tt_ref[...], st_ref[...])
    an = _layernorm(af1, ls_ref[...], lb_ref[...])
    af1_ref[...] = af1
    pc2_ref[...] = _mm(an, wc_ref[...]) + bc_ref[...]
    pn2_ref[...] = _mm(an, wn_ref[...]) + bn_ref[...]


def _conv2_body(af_ref, pc_ref, gv_ref, nbr_ref, sp_ref, wef_ref, bef_ref,
                wg_ref, bg_ref, wm_ref, bm_ref, tt_ref, st_ref, w1_ref,
                b1_ref, w2_ref, b2_ref, w3_ref, b3_ref, ja_ref, jb_ref,
                bja_ref, aij_ref, tbl_ref, ec_ref, *, d_o):
    af2 = _conv_accum(af_ref[...], pc_ref[...], gv_ref, nbr_ref,
                      wef_ref[...], bef_ref[...], wg_ref[...], bg_ref[...],
                      wm_ref[...], bm_ref[...], tt_ref[...], st_ref[...])
    h = _softplus(_mm(af2, w1_ref[...]) + b1_ref[...])
    h2 = _softplus(_mm(h, w2_ref[...]) + b2_ref[...])
    ec_ref[...] = _mm(h2, w3_ref[...]) + b3_ref[...]
    aij_ref[...] = _mm(af2, ja_ref[...]) + bja_ref[...]
    tbl_ref[:, 0:d_o] = _mm(af2, jb_ref[...])
    tbl_ref[:, d_o:] = sp_ref[...] * jnp.ones((1, d_o), jnp.float32)


def _edgej_body(aij_ref, gv_ref, nbr_ref, ec_ref, sp_ref, tai_ref, jcf_ref,
                wp_ref, ws_ref, wones_ref, bj2_ref, out_ref):
    @pl.when(pl.program_id(0) == 0)
    def _():
        out_ref[...] = jnp.zeros_like(out_ref)

    mh = (_mm(aij_ref[...], tai_ref[...]) + gv_ref[...]
          + _mm(nbr_ref[...], jcf_ref[...]))
    jh = _softplus(mh)
    jij = _mm(jh, wp_ref[...]) + bj2_ref[...]
    sj = _mm(mh, ws_ref[...])
    maskf = (_mm(nbr_ref[...], wones_ref[...]) > 0.0).astype(jnp.float32)
    accm = jnp.sum(jij * sj * maskf, axis=1, keepdims=True)
    es = ec_ref[...] + sp_ref[...] * accm
    out_ref[...] += jnp.sum(es, axis=0, keepdims=True)


def _sc_gather(table, idx_flat):
    B, = idx_flat.shape
    V, W = table.shape
    info = plsc.get_sparse_core_info()
    nw = info.num_cores * info.num_subcores
    bpw = B // nw
    chunk = _SC_CHUNK
    nchunks = bpw // chunk
    tail = bpw - nchunks * chunk
    assert nchunks % 3 == 0 and nchunks >= 6
    ntri = nchunks // 3
    mesh = plsc.VectorSubcoreMesh(core_axis_name="c", subcore_axis_name="s")

    scratch = (
        [pltpu.VMEM((chunk,), jnp.int32)] * 3
        + [pltpu.VMEM((chunk, W), jnp.float32)] * 3
        + [pltpu.SemaphoreType.DMA] * 9
    )
    if tail:
        scratch += [
            pltpu.VMEM((tail,), jnp.int32),
            pltpu.VMEM((tail, W), jnp.float32),
        ]

    @functools.partial(
        pl.kernel, mesh=mesh,
        out_type=jax.ShapeDtypeStruct((B, W), jnp.float32),
        compiler_params=pltpu.CompilerParams(use_tc_tiling_on_sc=False),
        scratch_types=scratch,
    )
    def gather_k(table_hbm, idx_hbm, out_hbm, i0, i1, i2, r0, r1, r2,
                 si0, si1, si2, sg0, sg1, sg2, so0, so1, so2, *tails):
        sid = lax.axis_index("s")
        cid = lax.axis_index("c")
        wid = sid * info.num_cores + cid
        base = wid * bpw
        ib = [i0, i1, i2]
        rb = [r0, r1, r2]
        si = [si0, si1, si2]
        sg = [sg0, sg1, sg2]
        so = [so0, so1, so2]

        def idx_start(k, b):
            pltpu.async_copy(idx_hbm.at[pl.ds(base + k * chunk, chunk)],
                             ib[b], si[b])

        def idx_wait(b):
            pltpu.make_async_copy(idx_hbm.at[pl.ds(base, chunk)], ib[b],
                                  si[b]).wait()

        def gat_wait(b):
            pltpu.make_async_copy(table_hbm.at[ib[b]], rb[b], sg[b]).wait()

        def out_wait(b):
            pltpu.make_async_copy(rb[b], out_hbm.at[pl.ds(base, chunk)],
                                  so[b]).wait()

        idx_start(0, 0)

        def body(c3, carry):
            k0 = 3 * c3

            def step(b, k, first, may_prefetch):
                idx_wait(b)

                @pl.when(c3 > 0)
                def _():
                    out_wait(b)

                pltpu.async_copy(table_hbm.at[ib[b]], rb[b], sg[b])
                nb = (b + 1) % 3
                if may_prefetch:
                    idx_start(k + 1, nb)
                else:
                    @pl.when(c3 + 1 < ntri)
                    def _():
                        idx_start(k + 1, nb)
                pb = (b + 2) % 3
                if first:
                    @pl.when(c3 > 0)
                    def _():
                        gat_wait(pb)
                        pltpu.async_copy(
                            rb[pb],
                            out_hbm.at[pl.ds(base + (k - 1) * chunk, chunk)],
                            so[pb])
                else:
                    gat_wait(pb)
                    pltpu.async_copy(
                        rb[pb],
                        out_hbm.at[pl.ds(base + (k - 1) * chunk, chunk)],
                        so[pb])

            step(0, k0, True, True)
            step(1, k0 + 1, False, True)
            step(2, k0 + 2, False, False)
            return carry

        lax.fori_loop(0, ntri, body, 0)
        gat_wait(2)
        pltpu.async_copy(
            rb[2], out_hbm.at[pl.ds(base + (nchunks - 1) * chunk, chunk)],
            so[2])
        if tail:
            idx_t, rows_t = tails
            off = base + nchunks * chunk
            pltpu.sync_copy(idx_hbm.at[pl.ds(off, tail)], idx_t)
            pltpu.async_copy(table_hbm.at[idx_t], rows_t, sg[0]).wait()
            pltpu.sync_copy(rows_t, out_hbm.at[pl.ds(off, tail)])
        out_wait(0)
        out_wait(1)
        out_wait(2)

    return gather_k(table, idx_flat)


def _row_spec(d):
    return pl.BlockSpec((_BN, d), lambda i: (i, 0))


def _full_spec(arr):
    nd = arr.ndim
    return pl.BlockSpec(arr.shape, lambda i, _nd=nd: (0,) * _nd)


def _tc_call(body, grid, ins, in_row_dims, out_shapes, out_row_dims):
    in_specs = [_row_spec(d) if d is not None else _full_spec(a)
                for a, d in zip(ins, in_row_dims)]
    out_specs = [_row_spec(d) if d is not None else
                 pl.BlockSpec(s.shape, lambda i: (0, 0))
                 for s, d in zip(out_shapes, out_row_dims)]
    return pl.pallas_call(
        body,
        grid=(grid,),
        in_specs=in_specs,
        out_specs=out_specs[0] if len(out_specs) == 1 else out_specs,
        out_shape=out_shapes[0] if len(out_shapes) == 1 else out_shapes,
    )(*ins)


def kernel(atom_fea, nbr_fea, nbr_fea_idx, atom_spins, params):
    N, M = nbr_fea_idx.shape
    d_e = nbr_fea.shape[-1]
    We, be = params["embed"]
    d_a = We.shape[1]
    Wne, bne = params["nbr_embed"]
    c1, c2 = params["convs"]
    ro1w, ro1b = params["ro1"]
    ro2w, ro2b = params["ro2"]
    ro3w, ro3b = params["ro3"]
    j1w, j1b = params["J1"]
    j2w, j2b = params["J2"]
    d_o = j1w.shape[1]
    wg3 = 2 * d_o

    f32 = jnp.float32
    r2 = lambda v: v.reshape(1, -1).astype(f32)

    wef1 = Wne @ c1["phi_e"][0]
    bef1 = bne @ c1["phi_e"][0] + c1["phi_e"][1]
    wef2 = Wne @ c2["phi_e"][0]
    bef2 = bne @ c2["phi_e"][0] + c2["phi_e"][1]
    j1a = j1w[:d_a]
    j1b_w = j1w[d_a:2 * d_a]
    j1c = Wne @ j1w[2 * d_a:]
    bj1f = j1b + bne @ j1w[2 * d_a:]

    eyeM = jnp.eye(M, dtype=jnp.float32)
    onesM = jnp.ones((1, M), jnp.float32)
    bd = lambda w: jnp.kron(eyeM, w.astype(jnp.float32))
    tl = lambda b: jnp.tile(b.astype(jnp.float32).reshape(-1), M)
    wef1_bd, bef1_t = bd(wef1), tl(bef1)
    wg1_bd, bg1_t = bd(c1["gate"][0]), tl(c1["gate"][1])
    wm1_bd, bm1_t = bd(c1["mag"][0]), tl(c1["mag"][1])
    wef2_bd, bef2_t = bd(wef2), tl(bef2)
    wg2_bd, bg2_t = bd(c2["gate"][0]), tl(c2["gate"][1])
    wm2_bd, bm2_t = bd(c2["mag"][0]), tl(c2["mag"][1])
    eyeA = jnp.eye(d_a, dtype=jnp.float32)
    tileT = jnp.kron(onesM, eyeA)
    sumT = jnp.kron(onesM.T, eyeA)
    zdo = jnp.zeros((d_o, d_o), jnp.float32)
    taiT = jnp.kron(onesM, jnp.concatenate([jnp.eye(d_o), zdo], 1))
    jcf_bd = bd(jnp.concatenate([j1c, jnp.zeros((d_e, d_o))], 1))
    wp_bd = bd(jnp.concatenate([j2w, jnp.zeros((d_o, 1))], 0))
    ws_bd = bd(jnp.concatenate([jnp.zeros((d_o, 1)),
                                jnp.full((d_o, 1), 1.0 / d_o)], 0))
    wones_bd = bd(jnp.ones((d_e, 1), jnp.float32))
    bj2_t = tl(j2b)

    nbr2d = nbr_fea.reshape(N, M * d_e)
    idx_flat = nbr_fea_idx.reshape(N * M).astype(jnp.int32)
    grid = N // _BN
    nodes = lambda d: jax.ShapeDtypeStruct((N, d), f32)

    af0, pc1, pn1 = _tc_call(
        _prep_body, grid,
        [atom_fea, We, r2(be), r2(c1["ln_scale"]), r2(c1["ln_bias"]),
         c1["phi_c"][0], r2(c1["phi_c"][1]), c1["phi_n"][0], r2(c1["phi_n"][1])],
        [atom_fea.shape[1], None, None, None, None, None, None, None, None],
        [nodes(d_a), nodes(d_a), nodes(d_a)], [d_a, d_a, d_a])

    g1 = _sc_gather(pn1, idx_flat).reshape(N, M * d_a)

    af1, pc2, pn2 = _tc_call(
        _conv1_body, grid,
        [af0, pc1, g1, nbr2d, wef1_bd, r2(bef1_t),
         wg1_bd, r2(bg1_t), wm1_bd, r2(bm1_t), tileT, sumT,
         r2(c2["ln_scale"]), r2(c2["ln_bias"]),
         c2["phi_c"][0], r2(c2["phi_c"][1]), c2["phi_n"][0], r2(c2["phi_n"][1])],
        [d_a, d_a, M * d_a, M * d_e] + [None] * 14,
        [nodes(d_a), nodes(d_a), nodes(d_a)], [d_a, d_a, d_a])

    g2 = _sc_gather(pn2, idx_flat).reshape(N, M * d_a)

    body_c = functools.partial(_conv2_body, d_o=d_o)
    aij, tbl, echem = _tc_call(
        body_c, grid,
        [af1, pc2, g2, nbr2d, atom_spins, wef2_bd, r2(bef2_t),
         wg2_bd, r2(bg2_t), wm2_bd, r2(bm2_t), tileT, sumT,
         ro1w, r2(ro1b), ro2w, r2(ro2b), ro3w, r2(ro3b),
         j1a, j1b_w, r2(bj1f)],
        [d_a, d_a, M * d_a, M * d_e, 1] + [None] * 17,
        [nodes(d_o), nodes(wg3), nodes(1)], [d_o, wg3, 1])

    g3 = _sc_gather(tbl, idx_flat).reshape(N, M * wg3)

    total = _tc_call(
        _edgej_body, grid,
        [aij, g3, nbr2d, echem, atom_spins, taiT, jcf_bd, wp_bd, ws_bd,
         wones_bd, r2(bj2_t)],
        [d_o, M * wg3, M * d_e, 1, 1] + [None] * 6,
        [jax.ShapeDtypeStruct((1, 1), f32)], [None])

    return total.reshape(())


# speedup vs baseline: 9.1885x; 1.0366x over previous
import functools

import jax
import jax.numpy as jnp
from jax import lax
from jax.experimental import pallas as pl
from jax.experimental.pallas import tpu as pltpu
from jax.experimental.pallas import tpu_sc as plsc

_BN = 2000
_SC_CHUNK = 128


def _softplus(x):
    return jnp.maximum(x, 0.0) + jnp.log(1.0 + jnp.exp(-jnp.abs(x)))


def _sigmoid(x):
    return 1.0 / (1.0 + jnp.exp(-x))


def _mm(a, b):
    return jnp.dot(a, b, preferred_element_type=jnp.float32)


def _layernorm(x, scale, bias, eps=1e-6):
    mu = jnp.mean(x, axis=1, keepdims=True)
    xc = x - mu
    var = jnp.mean(xc * xc, axis=1, keepdims=True)
    return xc * lax.rsqrt(var + eps) * scale + bias


def _conv_accum(af, pc, gv_ref, nbr_ref, wef, bef, wg, bg, wm, bm, tileT,
                sumT):
    pe = _mm(nbr_ref[...], wef) + bef
    pcs = _mm(pc, tileT)
    inter = pcs * gv_ref[...] * pe
    gate = _sigmoid(_mm(inter, wg) + bg)
    mag = _softplus(_mm(inter, wm) + bm)
    return af + _mm(gate * mag, sumT)


def _prep_body(x_ref, we_ref, be_ref, ls_ref, lb_ref, wc_ref, bc_ref,
               wn_ref, bn_ref, af0_ref, pc_ref, pn_ref):
    af0 = _mm(x_ref[...], we_ref[...]) + be_ref[...]
    an = _layernorm(af0, ls_ref[...], lb_ref[...])
    af0_ref[...] = af0
    pc_ref[...] = _mm(an, wc_ref[...]) + bc_ref[...]
    pn_ref[...] = _mm(an, wn_ref[...]) + bn_ref[...]


def _conv1_body(af_ref, pc_ref, gv_ref, nbr_ref, wef_ref, bef_ref, wg_ref,
                bg_ref, wm_ref, bm_ref, tt_ref, st_ref, ls_ref, lb_ref,
                wc_ref, bc_ref, wn_ref, bn_ref, af1_ref, pc2_ref, pn2_ref):
    af1 = _conv_accum(af_ref[...], pc_ref[...], gv_ref, nbr_ref,
                      wef_ref[...], bef_ref[...], wg_ref[...], bg_ref[...],
                      wm_ref[...], bm_ref[...], tt_ref[...], st_ref[...])
    an = _layernorm(af1, ls_ref[...], lb_ref[...])
    af1_ref[...] = af1
    pc2_ref[...] = _mm(an, wc_ref[...]) + bc_ref[...]
    pn2_ref[...] = _mm(an, wn_ref[...]) + bn_ref[...]


def _conv2_body(af_ref, pc_ref, gv_ref, nbr_ref, sp_ref, wef_ref, bef_ref,
                wg_ref, bg_ref, wm_ref, bm_ref, tt_ref, st_ref, w1_ref,
                b1_ref, w2_ref, b2_ref, w3_ref, b3_ref, ja_ref, jb_ref,
                bja_ref, aij_ref, tbl_ref, ec_ref, *, d_o):
    af2 = _conv_accum(af_ref[...], pc_ref[...], gv_ref, nbr_ref,
                      wef_ref[...], bef_ref[...], wg_ref[...], bg_ref[...],
                      wm_ref[...], bm_ref[...], tt_ref[...], st_ref[...])
    h = _softplus(_mm(af2, w1_ref[...]) + b1_ref[...])
    h2 = _softplus(_mm(h, w2_ref[...]) + b2_ref[...])
    ec_ref[...] = _mm(h2, w3_ref[...]) + b3_ref[...]
    aij_ref[...] = _mm(af2, ja_ref[...]) + bja_ref[...]
    tbl_ref[:, 0:d_o] = _mm(af2, jb_ref[...])
    tbl_ref[:, d_o:] = sp_ref[...] * jnp.ones((1, d_o), jnp.float32)


def _edgej_body(aij_ref, gv_ref, nbr_ref, ec_ref, sp_ref, tai_ref, jcf_ref,
                wp_ref, ws_ref, wones_ref, bj2_ref, out_ref):
    @pl.when(pl.program_id(0) == 0)
    def _():
        out_ref[...] = jnp.zeros_like(out_ref)

    mh = (_mm(aij_ref[...], tai_ref[...]) + gv_ref[...]
          + _mm(nbr_ref[...], jcf_ref[...]))
    jh = _softplus(mh)
    jij = _mm(jh, wp_ref[...]) + bj2_ref[...]
    sj = _mm(mh, ws_ref[...])
    maskf = (_mm(nbr_ref[...], wones_ref[...]) > 0.0).astype(jnp.float32)
    accm = jnp.sum(jij * sj * maskf, axis=1, keepdims=True)
    es = ec_ref[...] + sp_ref[...] * accm
    out_ref[...] += jnp.sum(es, axis=0, keepdims=True)


def _sc_gather(table, idx_flat):
    B, = idx_flat.shape
    V, W = table.shape
    info = plsc.get_sparse_core_info()
    nw = info.num_cores * info.num_subcores
    bpw = B // nw
    chunk = _SC_CHUNK
    nchunks = bpw // chunk
    tail = bpw - nchunks * chunk
    assert nchunks % 3 == 0 and nchunks >= 6
    ntri = nchunks // 3
    mesh = plsc.VectorSubcoreMesh(core_axis_name="c", subcore_axis_name="s")

    scratch = (
        [pltpu.VMEM((chunk,), jnp.int32)] * 3
        + [pltpu.VMEM((chunk, W), jnp.float32)] * 3
        + [pltpu.SemaphoreType.DMA] * 9
    )
    if tail:
        scratch += [
            pltpu.VMEM((tail,), jnp.int32),
            pltpu.VMEM((tail, W), jnp.float32),
        ]

    @functools.partial(
        pl.kernel, mesh=mesh,
        out_type=jax.ShapeDtypeStruct((B, W), jnp.float32),
        compiler_params=pltpu.CompilerParams(use_tc_tiling_on_sc=False),
        scratch_types=scratch,
    )
    def gather_k(table_hbm, idx_hbm, out_hbm, i0, i1, i2, r0, r1, r2,
                 si0, si1, si2, sg0, sg1, sg2, so0, so1, so2, *tails):
        sid = lax.axis_index("s")
        cid = lax.axis_index("c")
        wid = sid * info.num_cores + cid
        base = wid * bpw
        ib = [i0, i1, i2]
        rb = [r0, r1, r2]
        si = [si0, si1, si2]
        sg = [sg0, sg1, sg2]
        so = [so0, so1, so2]

        def idx_start(k, b):
            pltpu.async_copy(idx_hbm.at[pl.ds(base + k * chunk, chunk)],
                             ib[b], si[b])

        def idx_wait(b):
            pltpu.make_async_copy(idx_hbm.at[pl.ds(base, chunk)], ib[b],
                                  si[b]).wait()

        def gat_wait(b):
            pltpu.make_async_copy(table_hbm.at[ib[b]], rb[b], sg[b]).wait()

        def out_wait(b):
            pltpu.make_async_copy(rb[b], out_hbm.at[pl.ds(base, chunk)],
                                  so[b]).wait()

        idx_start(0, 0)

        def body(c3, carry):
            k0 = 3 * c3

            def step(b, k, first, may_prefetch):
                idx_wait(b)

                @pl.when(c3 > 0)
                def _():
                    out_wait(b)

                pltpu.async_copy(table_hbm.at[ib[b]], rb[b], sg[b])
                nb = (b + 1) % 3
                if may_prefetch:
                    idx_start(k + 1, nb)
                else:
                    @pl.when(c3 + 1 < ntri)
                    def _():
                        idx_start(k + 1, nb)
                pb = (b + 2) % 3
                if first:
                    @pl.when(c3 > 0)
                    def _():
                        gat_wait(pb)
                        pltpu.async_copy(
                            rb[pb],
                            out_hbm.at[pl.ds(base + (k - 1) * chunk, chunk)],
                            so[pb])
                else:
                    gat_wait(pb)
                    pltpu.async_copy(
                        rb[pb],
                        out_hbm.at[pl.ds(base + (k - 1) * chunk, chunk)],
                        so[pb])

            step(0, k0, True, True)
            step(1, k0 + 1, False, True)
            step(2, k0 + 2, False, False)
            return carry

        lax.fori_loop(0, ntri, body, 0)
        gat_wait(2)
        pltpu.async_copy(
            rb[2], out_hbm.at[pl.ds(base + (nchunks - 1) * chunk, chunk)],
            so[2])
        if tail:
            idx_t, rows_t = tails
            off = base + nchunks * chunk
            pltpu.sync_copy(idx_hbm.at[pl.ds(off, tail)], idx_t)
            pltpu.async_copy(table_hbm.at[idx_t], rows_t, sg[0]).wait()
            pltpu.sync_copy(rows_t, out_hbm.at[pl.ds(off, tail)])
        out_wait(0)
        out_wait(1)
        out_wait(2)

    return gather_k(table, idx_flat)


def _row_spec(d):
    return pl.BlockSpec((_BN, d), lambda i: (i, 0))


def _full_spec(arr):
    nd = arr.ndim
    return pl.BlockSpec(arr.shape, lambda i, _nd=nd: (0,) * _nd)


def _tc_call(body, grid, ins, in_row_dims, out_shapes, out_row_dims):
    in_specs = [_row_spec(d) if d is not None else _full_spec(a)
                for a, d in zip(ins, in_row_dims)]
    out_specs = [_row_spec(d) if d is not None else
                 pl.BlockSpec(s.shape, lambda i: (0, 0))
                 for s, d in zip(out_shapes, out_row_dims)]
    return pl.pallas_call(
        body,
        grid=(grid,),
        in_specs=in_specs,
        out_specs=out_specs[0] if len(out_specs) == 1 else out_specs,
        out_shape=out_shapes[0] if len(out_shapes) == 1 else out_shapes,
    )(*ins)


def kernel(atom_fea, nbr_fea, nbr_fea_idx, atom_spins, params):
    N, M = nbr_fea_idx.shape
    d_e = nbr_fea.shape[-1]
    We, be = params["embed"]
    d_a = We.shape[1]
    Wne, bne = params["nbr_embed"]
    c1, c2 = params["convs"]
    ro1w, ro1b = params["ro1"]
    ro2w, ro2b = params["ro2"]
    ro3w, ro3b = params["ro3"]
    j1w, j1b = params["J1"]
    j2w, j2b = params["J2"]
    d_o = j1w.shape[1]
    wg3 = 2 * d_o

    f32 = jnp.float32
    r2 = lambda v: v.reshape(1, -1).astype(f32)

    wef1 = Wne @ c1["phi_e"][0]
    bef1 = bne @ c1["phi_e"][0] + c1["phi_e"][1]
    wef2 = Wne @ c2["phi_e"][0]
    bef2 = bne @ c2["phi_e"][0] + c2["phi_e"][1]
    j1a = j1w[:d_a]
    j1b_w = j1w[d_a:2 * d_a]
    j1c = Wne @ j1w[2 * d_a:]
    bj1f = j1b + bne @ j1w[2 * d_a:]

    eyeM = jnp.eye(M, dtype=jnp.float32)
    onesM = jnp.ones((1, M), jnp.float32)
    bd = lambda w: jnp.kron(eyeM, w.astype(jnp.float32))
    tl = lambda b: jnp.tile(b.astype(jnp.float32).reshape(-1), M)
    wef1_bd, bef1_t = bd(wef1), tl(bef1)
    wg1_bd, bg1_t = bd(c1["gate"][0]), tl(c1["gate"][1])
    wm1_bd, bm1_t = bd(c1["mag"][0]), tl(c1["mag"][1])
    wef2_bd, bef2_t = bd(wef2), tl(bef2)
    wg2_bd, bg2_t = bd(c2["gate"][0]), tl(c2["gate"][1])
    wm2_bd, bm2_t = bd(c2["mag"][0]), tl(c2["mag"][1])
    eyeA = jnp.eye(d_a, dtype=jnp.float32)
    tileT = jnp.kron(onesM, eyeA)
    sumT = jnp.kron(onesM.T, eyeA)
    zdo = jnp.zeros((d_o, d_o), jnp.float32)
    taiT = jnp.kron(onesM, jnp.concatenate([jnp.eye(d_o), zdo], 1))
    jcf_bd = bd(jnp.concatenate([j1c, jnp.zeros((d_e, d_o))], 1))
    wp_bd = bd(jnp.concatenate([j2w, jnp.zeros((d_o, 1))], 0))
    ws_bd = bd(jnp.concatenate([jnp.zeros((d_o, 1)),
                                jnp.full((d_o, 1), 1.0 / d_o)], 0))
    wones_bd = bd(jnp.ones((d_e, 1), jnp.float32))
    bj2_t = tl(j2b)

    nbr2d = nbr_fea.reshape(N, M * d_e)
    idx_flat = nbr_fea_idx.reshape(N * M).astype(jnp.int32)
    grid = N // _BN
    nodes = lambda d: jax.ShapeDtypeStruct((N, d), f32)

    af0, pc1, pn1 = _tc_call(
        _prep_body, grid,
        [atom_fea, We, r2(be), r2(c1["ln_scale"]), r2(c1["ln_bias"]),
         c1["phi_c"][0], r2(c1["phi_c"][1]), c1["phi_n"][0], r2(c1["phi_n"][1])],
        [atom_fea.shape[1], None, None, None, None, None, None, None, None],
        [nodes(d_a), nodes(d_a), nodes(d_a)], [d_a, d_a, d_a])

    g1 = _sc_gather(pn1, idx_flat).reshape(N, M * d_a)

    af1, pc2, pn2 = _tc_call(
        _conv1_body, grid,
        [af0, pc1, g1, nbr2d, wef1_bd, r2(bef1_t),
         wg1_bd, r2(bg1_t), wm1_bd, r2(bm1_t), tileT, sumT,
         r2(c2["ln_scale"]), r2(c2["ln_bias"]),
         c2["phi_c"][0], r2(c2["phi_c"][1]), c2["phi_n"][0], r2(c2["phi_n"][1])],
        [d_a, d_a, M * d_a, M * d_e] + [None] * 14,
        [nodes(d_a), nodes(d_a), nodes(d_a)], [d_a, d_a, d_a])

    g2 = _sc_gather(pn2, idx_flat).reshape(N, M * d_a)

    body_c = functools.partial(_conv2_body, d_o=d_o)
    aij, tbl, echem = _tc_call(
        body_c, grid,
        [af1, pc2, g2, nbr2d, atom_spins, wef2_bd, r2(bef2_t),
         wg2_bd, r2(bg2_t), wm2_bd, r2(bm2_t), tileT, sumT,
         ro1w, r2(ro1b), ro2w, r2(ro2b), ro3w, r2(ro3b),
         j1a, j1b_w, r2(bj1f)],
        [d_a, d_a, M * d_a, M * d_e, 1] + [None] * 17,
        [nodes(d_o), nodes(wg3), nodes(1)], [d_o, wg3, 1])

    g3 = _sc_gather(tbl, idx_flat).reshape(N, M * wg3)

    total = _tc_call(
        _edgej_body, grid,
        [aij, g3, nbr2d, echem, atom_spins, taiT, jcf_bd, wp_bd, ws_bd,
         wones_bd, r2(bj2_t)],
        [d_o, M * wg3, M * d_e, 1, 1] + [None] * 6,
        [jax.ShapeDtypeStruct((1, 1), f32)], [None])

    return total.reshape(())


# speedup vs baseline: 9.5919x; 1.0439x over previous
import functools

import jax
import jax.numpy as jnp
from jax import lax
from jax.experimental import pallas as pl
from jax.experimental.pallas import tpu as pltpu
from jax.experimental.pallas import tpu_sc as plsc

_BN = 2000
_SC_CHUNK = 128


def _softplus(x):
    return jnp.maximum(x, 0.0) + jnp.log(1.0 + jnp.exp(-jnp.abs(x)))


def _sigmoid(x):
    return 1.0 / (1.0 + jnp.exp(-x))


def _mm(a, b):
    return jnp.dot(a, b, preferred_element_type=jnp.float32)


def _layernorm(x, scale, bias, eps=1e-6):
    mu = jnp.mean(x, axis=1, keepdims=True)
    xc = x - mu
    var = jnp.mean(xc * xc, axis=1, keepdims=True)
    return xc * lax.rsqrt(var + eps) * scale + bias


def _conv_accum(af, pc, gv_ref, nbr_ref, wef, bef, wg, bg, wm, bm, tileT,
                sumT):
    pe = _mm(nbr_ref[...], wef) + bef
    pcs = _mm(pc, tileT)
    inter = pcs * gv_ref[...] * pe
    gate = _sigmoid(_mm(inter, wg) + bg)
    mag = _softplus(_mm(inter, wm) + bm)
    return af + _mm(gate * mag, sumT)


def _prep_body(x_ref, we_ref, be_ref, ls_ref, lb_ref, wc_ref, bc_ref,
               wn_ref, bn_ref, af0_ref, pc_ref, pn_ref):
    af0 = _mm(x_ref[...], we_ref[...]) + be_ref[...]
    an = _layernorm(af0, ls_ref[...], lb_ref[...])
    af0_ref[...] = af0
    pc_ref[...] = _mm(an, wc_ref[...]) + bc_ref[...]
    pn_ref[...] = _mm(an, wn_ref[...]) + bn_ref[...]


def _conv1_body(af_ref, pc_ref, gv_ref, nbr_ref, wef_ref, bef_ref, wg_ref,
                bg_ref, wm_ref, bm_ref, tt_ref, st_ref, ls_ref, lb_ref,
                wc_ref, bc_ref, wn_ref, bn_ref, af1_ref, pc2_ref, pn2_ref):
    af1 = _conv_accum(af_ref[...], pc_ref[...], gv_ref, nbr_ref,
                      wef_ref[...], bef_ref[...], wg_ref[...], bg_ref[...],
                      wm_ref[...], bm_ref[...], tt_ref[...], st_ref[...])
    an = _layernorm(af1, ls_ref[...], lb_ref[...])
    af1_ref[...] = af1
    pc2_ref[...] = _mm(an, wc_ref[...]) + bc_ref[...]
    pn2_ref[...] = _mm(an, wn_ref[...]) + bn_ref[...]


def _conv2_body(af_ref, pc_ref, gv_ref, nbr_ref, wef_ref, bef_ref,
                wg_ref, bg_ref, wm_ref, bm_ref, tt_ref, st_ref, w1_ref,
                b1_ref, w2_ref, b2_ref, w3_ref, b3_ref, ja_ref, jb_ref,
                bja_ref, aij_ref, tbl_ref, ec_ref):
    af2 = _conv_accum(af_ref[...], pc_ref[...], gv_ref, nbr_ref,
                      wef_ref[...], bef_ref[...], wg_ref[...], bg_ref[...],
                      wm_ref[...], bm_ref[...], tt_ref[...], st_ref[...])
    h = _softplus(_mm(af2, w1_ref[...]) + b1_ref[...])
    h2 = _softplus(_mm(h, w2_ref[...]) + b2_ref[...])
    ec_ref[...] = _mm(h2, w3_ref[...]) + b3_ref[...]
    aij_ref[...] = _mm(af2, ja_ref[...]) + bja_ref[...]
    tbl_ref[...] = _mm(af2, jb_ref[...])


def _edgej_body(aij_ref, gv_ref, sj_ref, nbr_ref, ec_ref, sp_ref, tai_ref,
                jcf_ref, wp_ref, wones_ref, sel0_ref, bj2_ref, out_ref):
    @pl.when(pl.program_id(0) == 0)
    def _():
        out_ref[...] = jnp.zeros_like(out_ref)

    mh = (_mm(aij_ref[...], tai_ref[...]) + gv_ref[...]
          + _mm(nbr_ref[...], jcf_ref[...]))
    jh = _softplus(mh)
    jij = _mm(jh, wp_ref[...]) + bj2_ref[...]
    maskf = (_mm(nbr_ref[...], wones_ref[...]) > 0.0).astype(jnp.float32)
    sj = _mm(sj_ref[...], sel0_ref[...])
    accm = jnp.sum(jij * sj * maskf, axis=1, keepdims=True)
    es = ec_ref[...] + sp_ref[...] * accm
    out_ref[...] += jnp.sum(es, axis=0, keepdims=True)


def _sc_gather(table, idx_flat):
    B, = idx_flat.shape
    V, W = table.shape
    info = plsc.get_sparse_core_info()
    nw = info.num_cores * info.num_subcores
    bpw = B // nw
    chunk = _SC_CHUNK
    nchunks = bpw // chunk
    tail = bpw - nchunks * chunk
    assert nchunks % 3 == 0 and nchunks >= 6
    ntri = nchunks // 3
    mesh = plsc.VectorSubcoreMesh(core_axis_name="c", subcore_axis_name="s")

    scratch = (
        [pltpu.VMEM((chunk,), jnp.int32)] * 3
        + [pltpu.VMEM((chunk, W), jnp.float32)] * 3
        + [pltpu.SemaphoreType.DMA] * 9
    )
    if tail:
        scratch += [
            pltpu.VMEM((tail,), jnp.int32),
            pltpu.VMEM((tail, W), jnp.float32),
        ]

    @functools.partial(
        pl.kernel, mesh=mesh,
        out_type=jax.ShapeDtypeStruct((B, W), jnp.float32),
        compiler_params=pltpu.CompilerParams(use_tc_tiling_on_sc=False),
        scratch_types=scratch,
    )
    def gather_k(table_hbm, idx_hbm, out_hbm, i0, i1, i2, r0, r1, r2,
                 si0, si1, si2, sg0, sg1, sg2, so0, so1, so2, *tails):
        sid = lax.axis_index("s")
        cid = lax.axis_index("c")
        wid = sid * info.num_cores + cid
        base = wid * bpw
        ib = [i0, i1, i2]
        rb = [r0, r1, r2]
        si = [si0, si1, si2]
        sg = [sg0, sg1, sg2]
        so = [so0, so1, so2]

        def idx_start(k, b):
            pltpu.async_copy(idx_hbm.at[pl.ds(base + k * chunk, chunk)],
                             ib[b], si[b])

        def idx_wait(b):
            pltpu.make_async_copy(idx_hbm.at[pl.ds(base, chunk)], ib[b],
                                  si[b]).wait()

        def gat_wait(b):
            pltpu.make_async_copy(table_hbm.at[ib[b]], rb[b], sg[b]).wait()

        def out_wait(b):
            pltpu.make_async_copy(rb[b], out_hbm.at[pl.ds(base, chunk)],
                                  so[b]).wait()

        idx_start(0, 0)

        def body(c3, carry):
            k0 = 3 * c3

            def step(b, k, first, may_prefetch):
                idx_wait(b)

                @pl.when(c3 > 0)
                def _():
                    out_wait(b)

                pltpu.async_copy(table_hbm.at[ib[b]], rb[b], sg[b])
                nb = (b + 1) % 3
                if may_prefetch:
                    idx_start(k + 1, nb)
                else:
                    @pl.when(c3 + 1 < ntri)
                    def _():
                        idx_start(k + 1, nb)
                pb = (b + 2) % 3
                if first:
                    @pl.when(c3 > 0)
                    def _():
                        gat_wait(pb)
                        pltpu.async_copy(
                            rb[pb],
                            out_hbm.at[pl.ds(base + (k - 1) * chunk, chunk)],
                            so[pb])
                else:
                    gat_wait(pb)
                    pltpu.async_copy(
                        rb[pb],
                        out_hbm.at[pl.ds(base + (k - 1) * chunk, chunk)],
                        so[pb])

            step(0, k0, True, True)
            step(1, k0 + 1, False, True)
            step(2, k0 + 2, False, False)
            return carry

        lax.fori_loop(0, ntri, body, 0)
        gat_wait(2)
        pltpu.async_copy(
            rb[2], out_hbm.at[pl.ds(base + (nchunks - 1) * chunk, chunk)],
            so[2])
        if tail:
            idx_t, rows_t = tails
            off = base + nchunks * chunk
            pltpu.sync_copy(idx_hbm.at[pl.ds(off, tail)], idx_t)
            pltpu.async_copy(table_hbm.at[idx_t], rows_t, sg[0]).wait()
            pltpu.sync_copy(rows_t, out_hbm.at[pl.ds(off, tail)])
        out_wait(0)
        out_wait(1)
        out_wait(2)

    return gather_k(table, idx_flat)


def _row_spec(d):
    return pl.BlockSpec((_BN, d), lambda i: (i, 0))


def _full_spec(arr):
    nd = arr.ndim
    return pl.BlockSpec(arr.shape, lambda i, _nd=nd: (0,) * _nd)


def _tc_call(body, grid, ins, in_row_dims, out_shapes, out_row_dims):
    in_specs = [_row_spec(d) if d is not None else _full_spec(a)
                for a, d in zip(ins, in_row_dims)]
    out_specs = [_row_spec(d) if d is not None else
                 pl.BlockSpec(s.shape, lambda i: (0, 0))
                 for s, d in zip(out_shapes, out_row_dims)]
    return pl.pallas_call(
        body,
        grid=(grid,),
        in_specs=in_specs,
        out_specs=out_specs[0] if len(out_specs) == 1 else out_specs,
        out_shape=out_shapes[0] if len(out_shapes) == 1 else out_shapes,
    )(*ins)


def kernel(atom_fea, nbr_fea, nbr_fea_idx, atom_spins, params):
    N, M = nbr_fea_idx.shape
    d_e = nbr_fea.shape[-1]
    We, be = params["embed"]
    d_a = We.shape[1]
    Wne, bne = params["nbr_embed"]
    c1, c2 = params["convs"]
    ro1w, ro1b = params["ro1"]
    ro2w, ro2b = params["ro2"]
    ro3w, ro3b = params["ro3"]
    j1w, j1b = params["J1"]
    j2w, j2b = params["J2"]
    d_o = j1w.shape[1]
    wg3 = d_o

    f32 = jnp.float32
    r2 = lambda v: v.reshape(1, -1).astype(f32)

    wef1 = Wne @ c1["phi_e"][0]
    bef1 = bne @ c1["phi_e"][0] + c1["phi_e"][1]
    wef2 = Wne @ c2["phi_e"][0]
    bef2 = bne @ c2["phi_e"][0] + c2["phi_e"][1]
    j1a = j1w[:d_a]
    j1b_w = j1w[d_a:2 * d_a]
    j1c = Wne @ j1w[2 * d_a:]
    bj1f = j1b + bne @ j1w[2 * d_a:]

    eyeM = jnp.eye(M, dtype=jnp.float32)
    onesM = jnp.ones((1, M), jnp.float32)
    bd = lambda w: jnp.kron(eyeM, w.astype(jnp.float32))
    tl = lambda b: jnp.tile(b.astype(jnp.float32).reshape(-1), M)
    wef1_bd, bef1_t = bd(wef1), tl(bef1)
    wg1_bd, bg1_t = bd(c1["gate"][0]), tl(c1["gate"][1])
    wm1_bd, bm1_t = bd(c1["mag"][0]), tl(c1["mag"][1])
    wef2_bd, bef2_t = bd(wef2), tl(bef2)
    wg2_bd, bg2_t = bd(c2["gate"][0]), tl(c2["gate"][1])
    wm2_bd, bm2_t = bd(c2["mag"][0]), tl(c2["mag"][1])
    eyeA = jnp.eye(d_a, dtype=jnp.float32)
    tileT = jnp.kron(onesM, eyeA)
    sumT = jnp.kron(onesM.T, eyeA)
    taiT = jnp.kron(onesM, jnp.eye(d_o, dtype=jnp.float32))
    jcf_bd = bd(j1c)
    wp_bd = bd(j2w)
    wones_bd = bd(jnp.ones((d_e, 1), jnp.float32))
    bj2_t = tl(j2b)

    nbr2d = nbr_fea.reshape(N, M * d_e)
    idx_flat = nbr_fea_idx.reshape(N * M).astype(jnp.int32)
    grid = N // _BN
    nodes = lambda d: jax.ShapeDtypeStruct((N, d), f32)

    sel0 = bd(jnp.eye(16, 1, dtype=jnp.float32))
    g_sj = _sc_gather(jnp.tile(atom_spins.astype(f32), (1, 16)),
                      idx_flat).reshape(N, M * 16)

    af0, pc1, pn1 = _tc_call(
        _prep_body, grid,
        [atom_fea, We, r2(be), r2(c1["ln_scale"]), r2(c1["ln_bias"]),
         c1["phi_c"][0], r2(c1["phi_c"][1]), c1["phi_n"][0], r2(c1["phi_n"][1])],
        [atom_fea.shape[1], None, None, None, None, None, None, None, None],
        [nodes(d_a), nodes(d_a), nodes(d_a)], [d_a, d_a, d_a])

    g1 = _sc_gather(pn1, idx_flat).reshape(N, M * d_a)

    af1, pc2, pn2 = _tc_call(
        _conv1_body, grid,
        [af0, pc1, g1, nbr2d, wef1_bd, r2(bef1_t),
         wg1_bd, r2(bg1_t), wm1_bd, r2(bm1_t), tileT, sumT,
         r2(c2["ln_scale"]), r2(c2["ln_bias"]),
         c2["phi_c"][0], r2(c2["phi_c"][1]), c2["phi_n"][0], r2(c2["phi_n"][1])],
        [d_a, d_a, M * d_a, M * d_e] + [None] * 14,
        [nodes(d_a), nodes(d_a), nodes(d_a)], [d_a, d_a, d_a])

    g2 = _sc_gather(pn2, idx_flat).reshape(N, M * d_a)

    aij, tbl, echem = _tc_call(
        _conv2_body, grid,
        [af1, pc2, g2, nbr2d, wef2_bd, r2(bef2_t),
         wg2_bd, r2(bg2_t), wm2_bd, r2(bm2_t), tileT, sumT,
         ro1w, r2(ro1b), ro2w, r2(ro2b), ro3w, r2(ro3b),
         j1a, j1b_w, r2(bj1f)],
        [d_a, d_a, M * d_a, M * d_e] + [None] * 17,
        [nodes(d_o), nodes(wg3), nodes(1)], [d_o, wg3, 1])

    g3 = _sc_gather(tbl, idx_flat).reshape(N, M * wg3)

    total = _tc_call(
        _edgej_body, grid,
        [aij, g3, g_sj, nbr2d, echem, atom_spins, taiT, jcf_bd, wp_bd,
         wones_bd, sel0, r2(bj2_t)],
        [d_o, M * wg3, M * 16, M * d_e, 1, 1] + [None] * 6,
        [jax.ShapeDtypeStruct((1, 1), f32)], [None])

    return total.reshape(())
